# Initial kernel scaffold; baseline (speedup 1.0000x reference)
import jax
import jax.numpy as jnp
from jax.experimental import pallas as pl


def kernel(x, edge_index, node_type, edge_type, Wk0, Wq0, Wv0, ra0, rm0, rp0, Wa0, sk0, Wk1, Wq1, Wv1, ra1, rm1, rp1, Wa1, sk1):
    raise NotImplementedError("write your pallas kernel here")


# speedup vs baseline: 15.9531x; 15.9531x over previous
import functools
import math

import jax
import jax.numpy as jnp
from jax import lax
from jax.experimental import pallas as pl
from jax.experimental.pallas import tpu as pltpu
from jax.experimental.pallas import tpu_sc as plsc

NC = 2
NS = 16
NW = NC * NS
CH = 128

_HIGH = jax.lax.Precision.HIGHEST


def _dot(a, b):
  return jnp.dot(a, b, preferred_element_type=jnp.float32, precision=_HIGH)


def _acc_rows(n_nodes):
  rpt = (-(-n_nodes // NS) + 7) // 8 * 8
  return rpt, rpt * NS


def _proj_body(x_ref, nt8_ref, w_ref, o_ref):
  T = w_ref.shape[1]
  x = x_ref[...]
  o = jnp.zeros(o_ref.shape, jnp.float32)
  for t in range(T):
    xm = x * nt8_ref[:, t:t + 1]
    o = o + _dot(xm, w_ref[0, t])
  o_ref[...] = o


def _proj(x, nt8, w9, rows):
  n, d_in = x.shape
  nj, t_, _, d_out = w9.shape
  grid_i = n // rows
  return pl.pallas_call(
      _proj_body,
      grid=(grid_i, nj),
      in_specs=[
          pl.BlockSpec((rows, d_in), lambda i, j: (i, 0)),
          pl.BlockSpec((rows, 8), lambda i, j: (i, 0)),
          pl.BlockSpec((1, t_, d_in, d_out), lambda i, j: (j, 0, 0, 0)),
      ],
      out_specs=pl.BlockSpec((rows, d_out), lambda i, j: (j * grid_i + i, 0)),
      out_shape=jax.ShapeDtypeStruct((nj * n, d_out), jnp.float32),
  )(x, nt8, w9)


def _make_edge_kernel(n_nodes, n_chunks, heads, d, rw, ch, rn):
  hd = heads * d
  cpt = n_chunks // NW
  rpt, n_pad = _acc_rows(n_nodes)
  assert n_chunks % NW == 0 and cpt % 8 == 0
  nvec = d // 16
  voff = rn
  qoff = 2 * rn

  def body(table_hbm, kidx_hbm, dst_hbm, out_hbm,
           kidx_b, dst_b, vidx_b, qidx_b, kbuf, qbuf, vbuf, mbuf, zbuf,
           acc, sem0, sem1, sem2):
    c = lax.axis_index("c")
    s = lax.axis_index("s")
    wid = s * NC + c
    t_ch0 = wid * cpt

    for i in range(8):
      for j in range(rw // 16):
        zbuf[i, pl.ds(j * 16, 16)] = jnp.zeros((16,), jnp.float32)

    def zc(i, _):
      pltpu.sync_copy(zbuf, acc.at[pl.ds(s * rpt + i * 8, 8)])
      return 0
    lax.fori_loop(0, rpt // 8, zc, 0)
    plsc.subcore_barrier()

    lane = lax.iota(jnp.int32, 16)

    def batch_body(b, _):
      pltpu.sync_copy(kidx_hbm.at[pl.ds(t_ch0 + b * 8, 8)], kidx_b)
      pltpu.sync_copy(dst_hbm.at[pl.ds(t_ch0 + b * 8, 8)], dst_b)
      for r in range(8):
        for j in range(ch // 16):
          sl = pl.ds(j * 16, 16)
          vidx_b[r, sl] = kidx_b[r, sl] + voff
          qidx_b[r, sl] = dst_b[r, sl] + qoff

      def chunk_body(ci, _):
        cp1 = pltpu.async_copy(table_hbm.at[kidx_b.at[ci]], kbuf, sem0)
        cp2 = pltpu.async_copy(table_hbm.at[qidx_b.at[ci]], qbuf, sem1)
        cp3 = pltpu.async_copy(table_hbm.at[vidx_b.at[ci]], vbuf, sem2)
        cp1.wait()
        cp2.wait()
        cp3.wait()

        def edge_body(e, _):
          denrow = jnp.zeros((16,), jnp.float32)
          for h in range(heads):
            p = kbuf[e, pl.ds(h * d, 16)] * qbuf[e, pl.ds(h * d, 16)]
            for j in range(1, nvec):
              off = h * d + j * 16
              p = p + kbuf[e, pl.ds(off, 16)] * qbuf[e, pl.ds(off, 16)]
            traw = jnp.sum(p)
            exv = jnp.exp(jnp.full((16,), traw, jnp.float32))
            for j in range(nvec):
              off = h * d + j * 16
              mbuf[e, pl.ds(off, 16)] = exv * vbuf[e, pl.ds(off, 16)]
            denrow = denrow + jnp.where(lane == h, exv, 0.0)
          mbuf[e, pl.ds(hd, 16)] = denrow
          return 0

        lax.fori_loop(0, ch, edge_body, 0)
        pltpu.sync_copy(mbuf, acc.at[dst_b.at[ci]], add=True)
        return 0

      lax.fori_loop(0, 8, chunk_body, 0)
      return 0

    lax.fori_loop(0, cpt // 8, batch_body, 0)
    plsc.subcore_barrier()
    pltpu.sync_copy(acc.at[pl.ds(s * rpt, rpt)],
                    out_hbm.at[c, pl.ds(s * rpt, rpt)])

  mesh = plsc.VectorSubcoreMesh(core_axis_name="c", subcore_axis_name="s",
                                num_cores=NC, num_subcores=NS)
  return pl.kernel(
      body,
      out_type=jax.ShapeDtypeStruct((NC, n_pad, rw), jnp.float32),
      mesh=mesh,
      compiler_params=pltpu.CompilerParams(needs_layout_passes=False,
                                           use_tc_tiling_on_sc=False),
      scratch_types=[
          pltpu.VMEM((8, ch), jnp.int32),
          pltpu.VMEM((8, ch), jnp.int32),
          pltpu.VMEM((8, ch), jnp.int32),
          pltpu.VMEM((8, ch), jnp.int32),
          pltpu.VMEM((ch, hd), jnp.float32),
          pltpu.VMEM((ch, hd), jnp.float32),
          pltpu.VMEM((ch, hd), jnp.float32),
          pltpu.VMEM((ch, rw), jnp.float32),
          pltpu.VMEM((8, rw), jnp.float32),
          pltpu.VMEM_SHARED((n_pad, rw), jnp.float32),
          pltpu.SemaphoreType.DMA,
          pltpu.SemaphoreType.DMA,
          pltpu.SemaphoreType.DMA,
      ],
  )


def _comb_body(acc_ref, x_ref, nt8_ref, resw_ref, wa_ref, o_ref, heads, d,
               resid):
  T = wa_ref.shape[0]
  hd = heads * d
  s = acc_ref[0] + acc_ref[1]
  num = s[:, :hd]
  den = s[:, hd:hd + heads]
  inv = 1.0 / (den + 1e-9)
  if heads > 1:
    rowi = lax.broadcasted_iota(jnp.int32, (heads, hd), 0)
    coli = lax.broadcasted_iota(jnp.int32, (heads, hd), 1)
    sel = (coli // d == rowi).astype(jnp.float32)
    invwide = _dot(inv, sel)
  else:
    invwide = inv
  h_att = num * invwide
  o = jnp.zeros(o_ref.shape, jnp.float32)
  for t in range(T):
    hm = h_att * nt8_ref[:, t:t + 1]
    o = o + _dot(hm, wa_ref[t])
  if resid:
    o = o + x_ref[...] * resw_ref[...]
  o_ref[...] = o


def _comb(acc, x, nt8, resw, wa, heads, d, rows, resid):
  n = x.shape[0]
  rw = acc.shape[2]
  t_, _, d_out = wa.shape
  grid = n // rows
  body = functools.partial(_comb_body, heads=heads, d=d, resid=resid)
  return pl.pallas_call(
      body,
      grid=(grid,),
      in_specs=[
          pl.BlockSpec((NC, rows, rw), lambda i: (0, i, 0)),
          pl.BlockSpec((rows, x.shape[1]), lambda i: (i, 0)),
          pl.BlockSpec((rows, 8), lambda i: (i, 0)),
          pl.BlockSpec((rows, 1), lambda i: (i, 0)),
          pl.BlockSpec((t_, wa.shape[1], d_out), lambda i: (0, 0, 0)),
      ],
      out_specs=pl.BlockSpec((rows, d_out), lambda i: (i, 0)),
      out_shape=jax.ShapeDtypeStruct((n, d_out), jnp.float32),
  )(acc, x, nt8, resw, wa)


def kernel(x, edge_index, node_type, edge_type, Wk0, Wq0, Wv0, ra0, rm0, rp0,
           Wa0, sk0, Wk1, Wq1, Wv1, ra1, rm1, rp1, Wa1, sk1):
  n, d_in = x.shape
  e = edge_index.shape[1]
  t_num = Wk0.shape[0]
  r_num = ra0.shape[0]
  h0, hid = ra0.shape[1], ra0.shape[2]
  out_d = ra1.shape[2]
  _, n_pad = _acc_rows(n)

  src = edge_index[0]
  dst = edge_index[1]

  def fused_tables(wk, wv, wq, ra, rm, rp, heads, dh):
    din = wk.shape[1]
    wkr = wk.reshape(t_num, din, heads, dh)
    wvr = wv.reshape(t_num, din, heads, dh)
    scale = 1.0 / math.sqrt(dh)
    wkra = jnp.einsum("tihd,rhdf,rh->rtihf", wkr, ra, rp).reshape(
        r_num, t_num, din, heads * dh)
    wvrm = jnp.einsum("tihd,rhdf->rtihf", wvr, rm).reshape(
        r_num, t_num, din, heads * dh)
    return jnp.concatenate(
        [wkra, wvrm, (wq * scale)[None]], axis=0)

  w9_0 = fused_tables(Wk0, Wv0, Wq0, ra0, rm0, rp0, h0, hid)
  w9_1 = fused_tables(Wk1, Wv1, Wq1, ra1, rm1, rp1, 1, out_d)
  sig0 = jax.nn.sigmoid(sk0)
  wa0s = Wa0 * sig0[:, None, None]
  wa1s = Wa1 * jax.nn.sigmoid(sk1)[:, None, None]
  resw = (1.0 - sig0)[node_type][:, None]
  zero_resw = jnp.zeros((n, 1), jnp.float32)

  nt8 = (node_type[:, None] == jnp.arange(8, dtype=jnp.int32)[None, :]
         ).astype(jnp.float32)

  kidx = edge_type * n + src

  def chunked(ch):
    blk = NW * 8 * ch
    e_pad = ((e + blk - 1) // blk) * blk
    k2d = jnp.pad(kidx, [(0, e_pad - e)]).reshape(-1, ch)
    d2d = jnp.pad(dst, [(0, e_pad - e)],
                  constant_values=n_pad - 1).reshape(-1, ch)
    return k2d, d2d, e_pad // ch

  k2d0, d2d0, n_chunks0 = chunked(64)
  k2d1, d2d1, n_chunks1 = chunked(128)

  table0 = _proj(x, nt8, w9_0, rows=400)
  ek0 = _make_edge_kernel(n, n_chunks0, h0, hid, h0 * hid + 16, 64, r_num * n)
  acc0 = ek0(table0, k2d0, d2d0)
  h = _comb(acc0, x, nt8, resw, wa0s, h0, hid, rows=400, resid=True)

  table1 = _proj(h, nt8, w9_1, rows=400)
  ek1 = _make_edge_kernel(n, n_chunks1, 1, out_d, out_d + 16, 128, r_num * n)
  acc1 = ek1(table1, k2d1, d2d1)
  return _comb(acc1, h, nt8, zero_resw, wa1s, 1, out_d, rows=400, resid=False)


# speedup vs baseline: 21.9402x; 1.3753x over previous
import functools
import math

import jax
import jax.numpy as jnp
from jax import lax
from jax.experimental import pallas as pl
from jax.experimental.pallas import tpu as pltpu
from jax.experimental.pallas import tpu_sc as plsc

NC = 2
NS = 16
NW = NC * NS
CH = 128

_HIGH = jax.lax.Precision.HIGHEST


def _dot(a, b):
  return jnp.dot(a, b, preferred_element_type=jnp.float32, precision=_HIGH)


def _acc_rows(n_nodes):
  rpt = (-(-n_nodes // NS) + 7) // 8 * 8
  return rpt, rpt * NS


def _proj_body(x_ref, nt8_ref, w_ref, o_ref):
  T = w_ref.shape[1]
  x = x_ref[...]
  o = jnp.zeros(o_ref.shape, jnp.float32)
  for t in range(T):
    xm = x * nt8_ref[:, t:t + 1]
    o = o + _dot(xm, w_ref[0, t])
  o_ref[...] = o


def _proj(x, nt8, w9, rows):
  n, d_in = x.shape
  nj, t_, _, d_out = w9.shape
  grid_i = n // rows
  return pl.pallas_call(
      _proj_body,
      grid=(grid_i, nj),
      in_specs=[
          pl.BlockSpec((rows, d_in), lambda i, j: (i, 0)),
          pl.BlockSpec((rows, 8), lambda i, j: (i, 0)),
          pl.BlockSpec((1, t_, d_in, d_out), lambda i, j: (j, 0, 0, 0)),
      ],
      out_specs=pl.BlockSpec((rows, d_out), lambda i, j: (j * grid_i + i, 0)),
      out_shape=jax.ShapeDtypeStruct((nj * n, d_out), jnp.float32),
  )(x, nt8, w9)


def _make_edge_kernel(n_nodes, n_edges, n_chunks, heads, d, rw, ch, rn):
  hd = heads * d
  cpt = n_chunks // NW
  rpt, n_pad = _acc_rows(n_nodes)
  assert n_chunks % NW == 0 and cpt % 2 == 0
  nvec = d // 16
  voff = rn
  qoff = 2 * rn
  nv = -(-n_edges // ch)

  def body(table_hbm, kidx_hbm, dst_hbm, out_hbm,
           kidx_v, dst_v, vidx_r, qidx_r, kbuf, qbuf, vbuf, mbuf,
           acc, gsem0, gsem1):
    c = lax.axis_index("c")
    s = lax.axis_index("s")
    wid = s * NC + c
    t_ch0 = wid * cpt
    gsem = (gsem0, gsem1)

    for i in range(8):
      for j in range(rw // 16):
        mbuf[i, pl.ds(j * 16, 16)] = jnp.zeros((16,), jnp.float32)

    def zc(i, _):
      pltpu.sync_copy(mbuf.at[pl.ds(0, 8)], acc.at[pl.ds(s * rpt + i * 8, 8)])
      return 0
    lax.fori_loop(0, rpt // 8, zc, 0)
    plsc.subcore_barrier()

    pltpu.sync_copy(kidx_hbm.at[pl.ds(t_ch0, cpt)], kidx_v)
    pltpu.sync_copy(dst_hbm.at[pl.ds(t_ch0, cpt)], dst_v)

    lane = lax.iota(jnp.int32, 16)

    def chunk_ok(ci):
      return jnp.logical_and(t_ch0 + ci < nv, ci < cpt)

    def start(ci, par):
      for j in range(ch // 16):
        sl = pl.ds(j * 16, 16)
        vidx_r[par, sl] = kidx_v[ci, sl] + voff
        qidx_r[par, sl] = dst_v[ci, sl] + qoff
      pltpu.async_copy(table_hbm.at[kidx_v.at[ci]], kbuf.at[par], gsem[par])
      pltpu.async_copy(table_hbm.at[qidx_r.at[par]], qbuf.at[par], gsem[par])
      pltpu.async_copy(table_hbm.at[vidx_r.at[par]], vbuf.at[par], gsem[par])

    def finish(ci, par):
      dummy = table_hbm.at[pl.ds(0, ch)]
      pltpu.make_async_copy(dummy, kbuf.at[par], gsem[par]).wait()
      pltpu.make_async_copy(dummy, qbuf.at[par], gsem[par]).wait()
      pltpu.make_async_copy(dummy, vbuf.at[par], gsem[par]).wait()

      def edge_body(e, _):
        denrow = jnp.zeros((16,), jnp.float32)
        for h in range(heads):
          p = kbuf[par, e, pl.ds(h * d, 16)] * qbuf[par, e, pl.ds(h * d, 16)]
          for j in range(1, nvec):
            off = h * d + j * 16
            p = p + kbuf[par, e, pl.ds(off, 16)] * qbuf[par, e, pl.ds(off, 16)]
          traw = jnp.sum(p)
          exv = jnp.exp(jnp.full((16,), traw, jnp.float32))
          for j in range(nvec):
            off = h * d + j * 16
            mbuf[e, pl.ds(off, 16)] = exv * vbuf[par, e, pl.ds(off, 16)]
          denrow = denrow + jnp.where(lane == h, exv, 0.0)
        mbuf[e, pl.ds(hd, 16)] = denrow
        return 0

      lax.fori_loop(0, ch, edge_body, 0)
      pltpu.sync_copy(mbuf, acc.at[dst_v.at[ci]], add=True)

    @pl.when(chunk_ok(0))
    def _():
      start(0, 0)

    def pair_body(i2, _):
      ci0 = i2 * 2
      ci1 = ci0 + 1

      @pl.when(chunk_ok(ci1))
      def _():
        start(ci1, 1)

      @pl.when(chunk_ok(ci0))
      def _():
        finish(ci0, 0)

      @pl.when(chunk_ok(ci0 + 2))
      def _():
        start(ci0 + 2, 0)

      @pl.when(chunk_ok(ci1))
      def _():
        finish(ci1, 1)
      return 0

    lax.fori_loop(0, cpt // 2, pair_body, 0)
    plsc.subcore_barrier()
    pltpu.sync_copy(acc.at[pl.ds(s * rpt, rpt)],
                    out_hbm.at[c, pl.ds(s * rpt, rpt)])

  mesh = plsc.VectorSubcoreMesh(core_axis_name="c", subcore_axis_name="s",
                                num_cores=NC, num_subcores=NS)
  return pl.kernel(
      body,
      out_type=jax.ShapeDtypeStruct((NC, n_pad, rw), jnp.float32),
      mesh=mesh,
      compiler_params=pltpu.CompilerParams(needs_layout_passes=False,
                                           use_tc_tiling_on_sc=False),
      scratch_types=[
          pltpu.VMEM((cpt, ch), jnp.int32),
          pltpu.VMEM((cpt, ch), jnp.int32),
          pltpu.VMEM((2, ch), jnp.int32),
          pltpu.VMEM((2, ch), jnp.int32),
          pltpu.VMEM((2, ch, hd), jnp.float32),
          pltpu.VMEM((2, ch, hd), jnp.float32),
          pltpu.VMEM((2, ch, hd), jnp.float32),
          pltpu.VMEM((ch, rw), jnp.float32),
          pltpu.VMEM_SHARED((n_pad, rw), jnp.float32),
          pltpu.SemaphoreType.DMA,
          pltpu.SemaphoreType.DMA,
      ],
  )


def _comb_body(acc_ref, x_ref, nt8_ref, resw_ref, wa_ref, o_ref, heads, d,
               resid):
  T = wa_ref.shape[0]
  hd = heads * d
  s = acc_ref[0] + acc_ref[1]
  num = s[:, :hd]
  den = s[:, hd:hd + heads]
  inv = 1.0 / (den + 1e-9)
  if heads > 1:
    rowi = lax.broadcasted_iota(jnp.int32, (heads, hd), 0)
    coli = lax.broadcasted_iota(jnp.int32, (heads, hd), 1)
    sel = (coli // d == rowi).astype(jnp.float32)
    invwide = _dot(inv, sel)
  else:
    invwide = inv
  h_att = num * invwide
  o = jnp.zeros(o_ref.shape, jnp.float32)
  for t in range(T):
    hm = h_att * nt8_ref[:, t:t + 1]
    o = o + _dot(hm, wa_ref[t])
  if resid:
    o = o + x_ref[...] * resw_ref[...]
  o_ref[...] = o


def _comb(acc, x, nt8, resw, wa, heads, d, rows, resid):
  n = x.shape[0]
  rw = acc.shape[2]
  t_, _, d_out = wa.shape
  grid = n // rows
  body = functools.partial(_comb_body, heads=heads, d=d, resid=resid)
  return pl.pallas_call(
      body,
      grid=(grid,),
      in_specs=[
          pl.BlockSpec((NC, rows, rw), lambda i: (0, i, 0)),
          pl.BlockSpec((rows, x.shape[1]), lambda i: (i, 0)),
          pl.BlockSpec((rows, 8), lambda i: (i, 0)),
          pl.BlockSpec((rows, 1), lambda i: (i, 0)),
          pl.BlockSpec((t_, wa.shape[1], d_out), lambda i: (0, 0, 0)),
      ],
      out_specs=pl.BlockSpec((rows, d_out), lambda i: (i, 0)),
      out_shape=jax.ShapeDtypeStruct((n, d_out), jnp.float32),
  )(acc, x, nt8, resw, wa)


def kernel(x, edge_index, node_type, edge_type, Wk0, Wq0, Wv0, ra0, rm0, rp0,
           Wa0, sk0, Wk1, Wq1, Wv1, ra1, rm1, rp1, Wa1, sk1):
  n, d_in = x.shape
  e = edge_index.shape[1]
  t_num = Wk0.shape[0]
  r_num = ra0.shape[0]
  h0, hid = ra0.shape[1], ra0.shape[2]
  out_d = ra1.shape[2]
  _, n_pad = _acc_rows(n)

  src = edge_index[0]
  dst = edge_index[1]

  def fused_tables(wk, wv, wq, ra, rm, rp, heads, dh):
    din = wk.shape[1]
    wkr = wk.reshape(t_num, din, heads, dh)
    wvr = wv.reshape(t_num, din, heads, dh)
    scale = 1.0 / math.sqrt(dh)
    wkra = jnp.einsum("tihd,rhdf,rh->rtihf", wkr, ra, rp).reshape(
        r_num, t_num, din, heads * dh)
    wvrm = jnp.einsum("tihd,rhdf->rtihf", wvr, rm).reshape(
        r_num, t_num, din, heads * dh)
    return jnp.concatenate(
        [wkra, wvrm, (wq * scale)[None]], axis=0)

  w9_0 = fused_tables(Wk0, Wv0, Wq0, ra0, rm0, rp0, h0, hid)
  w9_1 = fused_tables(Wk1, Wv1, Wq1, ra1, rm1, rp1, 1, out_d)
  sig0 = jax.nn.sigmoid(sk0)
  wa0s = Wa0 * sig0[:, None, None]
  wa1s = Wa1 * jax.nn.sigmoid(sk1)[:, None, None]
  resw = (1.0 - sig0)[node_type][:, None]
  zero_resw = jnp.zeros((n, 1), jnp.float32)

  nt8 = (node_type[:, None] == jnp.arange(8, dtype=jnp.int32)[None, :]
         ).astype(jnp.float32)

  kidx = edge_type * n + src

  def chunked(ch):
    blk = NW * 8 * ch
    e_pad = ((e + blk - 1) // blk) * blk
    k2d = jnp.pad(kidx, [(0, e_pad - e)]).reshape(-1, ch)
    d2d = jnp.pad(dst, [(0, e_pad - e)],
                  constant_values=n_pad - 1).reshape(-1, ch)
    return k2d, d2d, e_pad // ch

  k2d0, d2d0, n_chunks0 = chunked(32)
  k2d1, d2d1, n_chunks1 = chunked(128)

  table0 = _proj(x, nt8, w9_0, rows=400)
  ek0 = _make_edge_kernel(n, e, n_chunks0, h0, hid, h0 * hid + 16, 32,
                          r_num * n)
  acc0 = ek0(table0, k2d0, d2d0)
  h = _comb(acc0, x, nt8, resw, wa0s, h0, hid, rows=400, resid=True)

  table1 = _proj(h, nt8, w9_1, rows=400)
  ek1 = _make_edge_kernel(n, e, n_chunks1, 1, out_d, out_d + 16, 128,
                          r_num * n)
  acc1 = ek1(table1, k2d1, d2d1)
  return _comb(acc1, h, nt8, zero_resw, wa1s, 1, out_d, rows=400, resid=False)


# speedup vs baseline: 25.4860x; 1.1616x over previous
import functools
import math

import jax
import jax.numpy as jnp
from jax import lax
from jax.experimental import pallas as pl
from jax.experimental.pallas import tpu as pltpu
from jax.experimental.pallas import tpu_sc as plsc

NC = 2
NS = 16
NW = NC * NS
CH = 128

_HIGH = jax.lax.Precision.HIGHEST


def _dot(a, b):
  return jnp.dot(a, b, preferred_element_type=jnp.float32, precision=_HIGH)


def _acc_rows(n_nodes):
  rpt = (-(-n_nodes // NS) + 7) // 8 * 8
  return rpt, rpt * NS


def _proj_body(x_ref, nt8_ref, w_ref, o_ref):
  nj, T = w_ref.shape[0], w_ref.shape[1]
  x = x_ref[...]
  xm = [x * nt8_ref[:, t:t + 1] for t in range(T)]
  for j in range(nj):
    o = _dot(xm[0], w_ref[j, 0])
    for t in range(1, T):
      o = o + _dot(xm[t], w_ref[j, t])
    o_ref[j] = o


def _proj(x, nt8, w9, rows):
  n, d_in = x.shape
  nj, t_, _, d_out = w9.shape
  grid_i = n // rows
  out = pl.pallas_call(
      _proj_body,
      grid=(grid_i,),
      in_specs=[
          pl.BlockSpec((rows, d_in), lambda i: (i, 0)),
          pl.BlockSpec((rows, 8), lambda i: (i, 0)),
          pl.BlockSpec((nj, t_, d_in, d_out), lambda i: (0, 0, 0, 0)),
      ],
      out_specs=pl.BlockSpec((nj, rows, d_out), lambda i: (0, i, 0)),
      out_shape=jax.ShapeDtypeStruct((nj, n, d_out), jnp.float32),
  )(x, nt8, w9)
  return out.reshape(nj * n, d_out)


def _make_edge_kernel(n_nodes, n_edges, n_chunks, heads, d, rw, ch, rn):
  hd = heads * d
  cpt = n_chunks // NW
  rpt, n_pad = _acc_rows(n_nodes)
  assert n_chunks % NW == 0 and cpt % 2 == 0
  nvec = d // 16
  voff = rn
  qoff = 2 * rn
  nv = -(-n_edges // ch)

  def body(table_hbm, kidx_hbm, dst_hbm, out_hbm,
           kidx_v, dst_v, vidx_r, qidx_r, kbuf, qbuf, vbuf, mbuf,
           acc, gsem0, gsem1):
    c = lax.axis_index("c")
    s = lax.axis_index("s")
    wid = s * NC + c
    t_ch0 = wid * cpt
    gsem = (gsem0, gsem1)

    for i in range(8):
      for j in range(rw // 16):
        mbuf[i, pl.ds(j * 16, 16)] = jnp.zeros((16,), jnp.float32)

    def zc(i, _):
      pltpu.sync_copy(mbuf.at[pl.ds(0, 8)], acc.at[pl.ds(s * rpt + i * 8, 8)])
      return 0
    lax.fori_loop(0, rpt // 8, zc, 0)
    plsc.subcore_barrier()

    pltpu.sync_copy(kidx_hbm.at[pl.ds(t_ch0, cpt)], kidx_v)
    pltpu.sync_copy(dst_hbm.at[pl.ds(t_ch0, cpt)], dst_v)

    lane = lax.iota(jnp.int32, 16)

    def chunk_ok(ci):
      return jnp.logical_and(t_ch0 + ci < nv, ci < cpt)

    def start(ci, par):
      for j in range(ch // 16):
        sl = pl.ds(j * 16, 16)
        vidx_r[par, sl] = kidx_v[ci, sl] + voff
        qidx_r[par, sl] = dst_v[ci, sl] + qoff
      pltpu.async_copy(table_hbm.at[kidx_v.at[ci]], kbuf.at[par], gsem[par])
      pltpu.async_copy(table_hbm.at[qidx_r.at[par]], qbuf.at[par], gsem[par])
      pltpu.async_copy(table_hbm.at[vidx_r.at[par]], vbuf.at[par], gsem[par])

    def finish(ci, par):
      dummy = table_hbm.at[pl.ds(0, ch)]
      pltpu.make_async_copy(dummy, kbuf.at[par], gsem[par]).wait()
      pltpu.make_async_copy(dummy, qbuf.at[par], gsem[par]).wait()
      pltpu.make_async_copy(dummy, vbuf.at[par], gsem[par]).wait()

      def edge_body(e, _):
        denrow = jnp.zeros((16,), jnp.float32)
        for h in range(heads):
          p = kbuf[par, e, pl.ds(h * d, 16)] * qbuf[par, e, pl.ds(h * d, 16)]
          for j in range(1, nvec):
            off = h * d + j * 16
            p = p + kbuf[par, e, pl.ds(off, 16)] * qbuf[par, e, pl.ds(off, 16)]
          traw = jnp.sum(p)
          exv = jnp.exp(jnp.full((16,), traw, jnp.float32))
          for j in range(nvec):
            off = h * d + j * 16
            mbuf[e, pl.ds(off, 16)] = exv * vbuf[par, e, pl.ds(off, 16)]
          denrow = denrow + jnp.where(lane == h, exv, 0.0)
        mbuf[e, pl.ds(hd, 16)] = denrow
        return 0

      lax.fori_loop(0, ch, edge_body, 0)
      pltpu.sync_copy(mbuf, acc.at[dst_v.at[ci]], add=True)

    @pl.when(chunk_ok(0))
    def _():
      start(0, 0)

    def pair_body(i2, _):
      ci0 = i2 * 2
      ci1 = ci0 + 1

      @pl.when(chunk_ok(ci1))
      def _():
        start(ci1, 1)

      @pl.when(chunk_ok(ci0))
      def _():
        finish(ci0, 0)

      @pl.when(chunk_ok(ci0 + 2))
      def _():
        start(ci0 + 2, 0)

      @pl.when(chunk_ok(ci1))
      def _():
        finish(ci1, 1)
      return 0

    lax.fori_loop(0, cpt // 2, pair_body, 0)
    plsc.subcore_barrier()
    pltpu.sync_copy(acc.at[pl.ds(s * rpt, rpt)],
                    out_hbm.at[c, pl.ds(s * rpt, rpt)])

  mesh = plsc.VectorSubcoreMesh(core_axis_name="c", subcore_axis_name="s",
                                num_cores=NC, num_subcores=NS)
  return pl.kernel(
      body,
      out_type=jax.ShapeDtypeStruct((NC, n_pad, rw), jnp.float32),
      mesh=mesh,
      compiler_params=pltpu.CompilerParams(needs_layout_passes=False,
                                           use_tc_tiling_on_sc=False),
      scratch_types=[
          pltpu.VMEM((cpt, ch), jnp.int32),
          pltpu.VMEM((cpt, ch), jnp.int32),
          pltpu.VMEM((2, ch), jnp.int32),
          pltpu.VMEM((2, ch), jnp.int32),
          pltpu.VMEM((2, ch, hd), jnp.float32),
          pltpu.VMEM((2, ch, hd), jnp.float32),
          pltpu.VMEM((2, ch, hd), jnp.float32),
          pltpu.VMEM((ch, rw), jnp.float32),
          pltpu.VMEM_SHARED((n_pad, rw), jnp.float32),
          pltpu.SemaphoreType.DMA,
          pltpu.SemaphoreType.DMA,
      ],
  )


def _comb_body(acc_ref, x_ref, nt8_ref, resw_ref, wa_ref, o_ref, heads, d,
               resid):
  T = wa_ref.shape[0]
  hd = heads * d
  s = acc_ref[0] + acc_ref[1]
  num = s[:, :hd]
  den = s[:, hd:hd + heads]
  inv = 1.0 / (den + 1e-9)
  if heads > 1:
    rowi = lax.broadcasted_iota(jnp.int32, (heads, hd), 0)
    coli = lax.broadcasted_iota(jnp.int32, (heads, hd), 1)
    sel = (coli // d == rowi).astype(jnp.float32)
    invwide = _dot(inv, sel)
  else:
    invwide = inv
  h_att = num * invwide
  o = jnp.zeros(o_ref.shape, jnp.float32)
  for t in range(T):
    hm = h_att * nt8_ref[:, t:t + 1]
    o = o + _dot(hm, wa_ref[t])
  if resid:
    o = o + x_ref[...] * resw_ref[...]
  o_ref[...] = o


def _comb(acc, x, nt8, resw, wa, heads, d, rows, resid):
  n = x.shape[0]
  rw = acc.shape[2]
  t_, _, d_out = wa.shape
  grid = n // rows
  body = functools.partial(_comb_body, heads=heads, d=d, resid=resid)
  return pl.pallas_call(
      body,
      grid=(grid,),
      in_specs=[
          pl.BlockSpec((NC, rows, rw), lambda i: (0, i, 0)),
          pl.BlockSpec((rows, x.shape[1]), lambda i: (i, 0)),
          pl.BlockSpec((rows, 8), lambda i: (i, 0)),
          pl.BlockSpec((rows, 1), lambda i: (i, 0)),
          pl.BlockSpec((t_, wa.shape[1], d_out), lambda i: (0, 0, 0)),
      ],
      out_specs=pl.BlockSpec((rows, d_out), lambda i: (i, 0)),
      out_shape=jax.ShapeDtypeStruct((n, d_out), jnp.float32),
  )(acc, x, nt8, resw, wa)


def kernel(x, edge_index, node_type, edge_type, Wk0, Wq0, Wv0, ra0, rm0, rp0,
           Wa0, sk0, Wk1, Wq1, Wv1, ra1, rm1, rp1, Wa1, sk1):
  n, d_in = x.shape
  e = edge_index.shape[1]
  t_num = Wk0.shape[0]
  r_num = ra0.shape[0]
  h0, hid = ra0.shape[1], ra0.shape[2]
  out_d = ra1.shape[2]
  _, n_pad = _acc_rows(n)

  src = edge_index[0]
  dst = edge_index[1]

  def fused_tables(wk, wv, wq, ra, rm, rp, heads, dh):
    din = wk.shape[1]
    wkr = wk.reshape(t_num, din, heads, dh)
    wvr = wv.reshape(t_num, din, heads, dh)
    scale = 1.0 / math.sqrt(dh)
    wkra = jnp.einsum("tihd,rhdf,rh->rtihf", wkr, ra, rp).reshape(
        r_num, t_num, din, heads * dh)
    wvrm = jnp.einsum("tihd,rhdf->rtihf", wvr, rm).reshape(
        r_num, t_num, din, heads * dh)
    return jnp.concatenate(
        [wkra, wvrm, (wq * scale)[None]], axis=0)

  w9_0 = fused_tables(Wk0, Wv0, Wq0, ra0, rm0, rp0, h0, hid)
  w9_1 = fused_tables(Wk1, Wv1, Wq1, ra1, rm1, rp1, 1, out_d)
  sig0 = jax.nn.sigmoid(sk0)
  wa0s = Wa0 * sig0[:, None, None]
  wa1s = Wa1 * jax.nn.sigmoid(sk1)[:, None, None]
  resw = (1.0 - sig0)[node_type][:, None]
  zero_resw = jnp.zeros((n, 1), jnp.float32)

  nt8 = (node_type[:, None] == jnp.arange(8, dtype=jnp.int32)[None, :]
         ).astype(jnp.float32)

  kidx = edge_type * n + src

  def chunked(ch):
    blk = NW * 8 * ch
    e_pad = ((e + blk - 1) // blk) * blk
    k2d = jnp.pad(kidx, [(0, e_pad - e)]).reshape(-1, ch)
    d2d = jnp.pad(dst, [(0, e_pad - e)],
                  constant_values=n_pad - 1).reshape(-1, ch)
    return k2d, d2d, e_pad // ch

  k2d0, d2d0, n_chunks0 = chunked(32)
  k2d1, d2d1, n_chunks1 = chunked(128)

  table0 = _proj(x, nt8, w9_0, rows=400)
  ek0 = _make_edge_kernel(n, e, n_chunks0, h0, hid, h0 * hid + 16, 32,
                          r_num * n)
  acc0 = ek0(table0, k2d0, d2d0)
  h = _comb(acc0, x, nt8, resw, wa0s, h0, hid, rows=400, resid=True)

  table1 = _proj(h, nt8, w9_1, rows=400)
  ek1 = _make_edge_kernel(n, e, n_chunks1, 1, out_d, out_d + 16, 128,
                          r_num * n)
  acc1 = ek1(table1, k2d1, d2d1)
  return _comb(acc1, h, nt8, zero_resw, wa1s, 1, out_d, rows=400, resid=False)


# speedup vs baseline: 33.3575x; 1.3089x over previous
import functools
import math

import jax
import jax.numpy as jnp
from jax import lax
from jax.experimental import pallas as pl
from jax.experimental.pallas import tpu as pltpu
from jax.experimental.pallas import tpu_sc as plsc

NC = 2
NS = 16
NW = NC * NS
CH = 128

_HIGH = jax.lax.Precision.HIGHEST


def _dot(a, b):
  return jnp.dot(a, b, preferred_element_type=jnp.float32, precision=_HIGH)


def _acc_rows(n_nodes):
  rpt = (-(-n_nodes // NS) + 7) // 8 * 8
  return rpt, rpt * NS


def _proj_body(x_ref, nt8_ref, w_ref, o_ref):
  nj, T = w_ref.shape[0], w_ref.shape[1]
  x = x_ref[...]
  xm = [x * nt8_ref[:, t:t + 1] for t in range(T)]
  for j in range(nj):
    o = _dot(xm[0], w_ref[j, 0])
    for t in range(1, T):
      o = o + _dot(xm[t], w_ref[j, t])
    o_ref[j] = o


def _proj(x, nt8, w9, rows):
  n, d_in = x.shape
  nj, t_, _, d_out = w9.shape
  grid_i = n // rows
  out = pl.pallas_call(
      _proj_body,
      grid=(grid_i,),
      in_specs=[
          pl.BlockSpec((rows, d_in), lambda i: (i, 0)),
          pl.BlockSpec((rows, 8), lambda i: (i, 0)),
          pl.BlockSpec((nj, t_, d_in, d_out), lambda i: (0, 0, 0, 0)),
      ],
      out_specs=pl.BlockSpec((nj, rows, d_out), lambda i: (0, i, 0)),
      out_shape=jax.ShapeDtypeStruct((nj, n, d_out), jnp.float32),
  )(x, nt8, w9)
  return out.reshape(nj * n, d_out)


def _make_edge_kernel(n_nodes, n_edges, n_chunks, heads, d, rw, ch, rn):
  hd = heads * d
  cpt = n_chunks // NW
  rpt, n_pad = _acc_rows(n_nodes)
  assert n_chunks % NW == 0 and cpt % 2 == 0
  nvec = d // 16
  voff = rn
  qoff = 2 * rn
  nv = -(-n_edges // ch)

  def body(table_hbm, kidx_hbm, dst_hbm, out_hbm,
           kidx_v, dst_v, vidx_r, qidx_r, kbuf, qbuf, vbuf, mbuf,
           acc, gsem0, gsem1):
    c = lax.axis_index("c")
    s = lax.axis_index("s")
    wid = s * NC + c
    t_ch0 = wid * cpt
    gsem = (gsem0, gsem1)

    for i in range(8):
      for j in range(rw // 16):
        mbuf[i, pl.ds(j * 16, 16)] = jnp.zeros((16,), jnp.float32)

    def zc(i, _):
      pltpu.sync_copy(mbuf.at[pl.ds(0, 8)], acc.at[pl.ds(s * rpt + i * 8, 8)])
      return 0
    lax.fori_loop(0, rpt // 8, zc, 0)
    plsc.subcore_barrier()

    pltpu.sync_copy(kidx_hbm.at[pl.ds(t_ch0, cpt)], kidx_v)
    pltpu.sync_copy(dst_hbm.at[pl.ds(t_ch0, cpt)], dst_v)

    lane = lax.iota(jnp.int32, 16)

    def chunk_ok(ci):
      return jnp.logical_and(t_ch0 + ci < nv, ci < cpt)

    def start(ci, par):
      for j in range(ch // 16):
        sl = pl.ds(j * 16, 16)
        vidx_r[par, sl] = kidx_v[ci, sl] + voff
        qidx_r[par, sl] = dst_v[ci, sl] + qoff
      pltpu.async_copy(table_hbm.at[kidx_v.at[ci]], kbuf.at[par], gsem[par])
      pltpu.async_copy(table_hbm.at[qidx_r.at[par]], qbuf.at[par], gsem[par])
      pltpu.async_copy(table_hbm.at[vidx_r.at[par]], vbuf.at[par], gsem[par])

    def finish(ci, par):
      dummy = table_hbm.at[pl.ds(0, ch)]
      pltpu.make_async_copy(dummy, kbuf.at[par], gsem[par]).wait()
      pltpu.make_async_copy(dummy, qbuf.at[par], gsem[par]).wait()
      pltpu.make_async_copy(dummy, vbuf.at[par], gsem[par]).wait()

      def one_edge(e):
        traws = []
        for h in range(heads):
          p = kbuf[par, e, pl.ds(h * d, 16)] * qbuf[par, e, pl.ds(h * d, 16)]
          for j in range(1, nvec):
            off = h * d + j * 16
            p = p + kbuf[par, e, pl.ds(off, 16)] * qbuf[par, e, pl.ds(off, 16)]
          traws.append(jnp.sum(p))
        denrow = jnp.zeros((16,), jnp.float32)
        for h in range(heads):
          exv = jnp.exp(jnp.full((16,), traws[h], jnp.float32))
          for j in range(nvec):
            off = h * d + j * 16
            mbuf[e, pl.ds(off, 16)] = exv * vbuf[par, e, pl.ds(off, 16)]
          denrow = denrow + jnp.where(lane == h, exv, 0.0)
        mbuf[e, pl.ds(hd, 16)] = denrow

      def edge_body(e2, _):
        one_edge(e2 * 2)
        one_edge(e2 * 2 + 1)
        return 0

      lax.fori_loop(0, ch // 2, edge_body, 0)
      pltpu.sync_copy(mbuf, acc.at[dst_v.at[ci]], add=True)

    @pl.when(chunk_ok(0))
    def _():
      start(0, 0)

    def pair_body(i2, _):
      ci0 = i2 * 2
      ci1 = ci0 + 1

      @pl.when(chunk_ok(ci1))
      def _():
        start(ci1, 1)

      @pl.when(chunk_ok(ci0))
      def _():
        finish(ci0, 0)

      @pl.when(chunk_ok(ci0 + 2))
      def _():
        start(ci0 + 2, 0)

      @pl.when(chunk_ok(ci1))
      def _():
        finish(ci1, 1)
      return 0

    lax.fori_loop(0, cpt // 2, pair_body, 0)
    plsc.subcore_barrier()
    pltpu.sync_copy(acc.at[pl.ds(s * rpt, rpt)],
                    out_hbm.at[c, pl.ds(s * rpt, rpt)])

  mesh = plsc.VectorSubcoreMesh(core_axis_name="c", subcore_axis_name="s",
                                num_cores=NC, num_subcores=NS)
  return pl.kernel(
      body,
      out_type=jax.ShapeDtypeStruct((NC, n_pad, rw), jnp.float32),
      mesh=mesh,
      compiler_params=pltpu.CompilerParams(needs_layout_passes=False,
                                           use_tc_tiling_on_sc=False),
      scratch_types=[
          pltpu.VMEM((cpt, ch), jnp.int32),
          pltpu.VMEM((cpt, ch), jnp.int32),
          pltpu.VMEM((2, ch), jnp.int32),
          pltpu.VMEM((2, ch), jnp.int32),
          pltpu.VMEM((2, ch, hd), jnp.float32),
          pltpu.VMEM((2, ch, hd), jnp.float32),
          pltpu.VMEM((2, ch, hd), jnp.float32),
          pltpu.VMEM((ch, rw), jnp.float32),
          pltpu.VMEM_SHARED((n_pad, rw), jnp.float32),
          pltpu.SemaphoreType.DMA,
          pltpu.SemaphoreType.DMA,
      ],
  )


def _comb_body(acc_ref, x_ref, nt8_ref, resw_ref, wa_ref, o_ref, heads, d,
               resid):
  T = wa_ref.shape[0]
  hd = heads * d
  s = acc_ref[0] + acc_ref[1]
  num = s[:, :hd]
  den = s[:, hd:hd + heads]
  inv = 1.0 / (den + 1e-9)
  if heads > 1:
    rowi = lax.broadcasted_iota(jnp.int32, (heads, hd), 0)
    coli = lax.broadcasted_iota(jnp.int32, (heads, hd), 1)
    sel = (coli // d == rowi).astype(jnp.float32)
    invwide = _dot(inv, sel)
  else:
    invwide = inv
  h_att = num * invwide
  o = jnp.zeros(o_ref.shape, jnp.float32)
  for t in range(T):
    hm = h_att * nt8_ref[:, t:t + 1]
    o = o + _dot(hm, wa_ref[t])
  if resid:
    o = o + x_ref[...] * resw_ref[...]
  o_ref[...] = o


def _comb(acc, x, nt8, resw, wa, heads, d, rows, resid):
  n = x.shape[0]
  rw = acc.shape[2]
  t_, _, d_out = wa.shape
  grid = n // rows
  body = functools.partial(_comb_body, heads=heads, d=d, resid=resid)
  return pl.pallas_call(
      body,
      grid=(grid,),
      in_specs=[
          pl.BlockSpec((NC, rows, rw), lambda i: (0, i, 0)),
          pl.BlockSpec((rows, x.shape[1]), lambda i: (i, 0)),
          pl.BlockSpec((rows, 8), lambda i: (i, 0)),
          pl.BlockSpec((rows, 1), lambda i: (i, 0)),
          pl.BlockSpec((t_, wa.shape[1], d_out), lambda i: (0, 0, 0)),
      ],
      out_specs=pl.BlockSpec((rows, d_out), lambda i: (i, 0)),
      out_shape=jax.ShapeDtypeStruct((n, d_out), jnp.float32),
  )(acc, x, nt8, resw, wa)


def kernel(x, edge_index, node_type, edge_type, Wk0, Wq0, Wv0, ra0, rm0, rp0,
           Wa0, sk0, Wk1, Wq1, Wv1, ra1, rm1, rp1, Wa1, sk1):
  n, d_in = x.shape
  e = edge_index.shape[1]
  t_num = Wk0.shape[0]
  r_num = ra0.shape[0]
  h0, hid = ra0.shape[1], ra0.shape[2]
  out_d = ra1.shape[2]
  _, n_pad = _acc_rows(n)

  src = edge_index[0]
  dst = edge_index[1]

  def fused_tables(wk, wv, wq, ra, rm, rp, heads, dh):
    din = wk.shape[1]
    wkr = wk.reshape(t_num, din, heads, dh)
    wvr = wv.reshape(t_num, din, heads, dh)
    scale = 1.0 / math.sqrt(dh)
    wkra = jnp.einsum("tihd,rhdf,rh->rtihf", wkr, ra, rp).reshape(
        r_num, t_num, din, heads * dh)
    wvrm = jnp.einsum("tihd,rhdf->rtihf", wvr, rm).reshape(
        r_num, t_num, din, heads * dh)
    return jnp.concatenate(
        [wkra, wvrm, (wq * scale)[None]], axis=0)

  w9_0 = fused_tables(Wk0, Wv0, Wq0, ra0, rm0, rp0, h0, hid)
  w9_1 = fused_tables(Wk1, Wv1, Wq1, ra1, rm1, rp1, 1, out_d)
  sig0 = jax.nn.sigmoid(sk0)
  wa0s = Wa0 * sig0[:, None, None]
  wa1s = Wa1 * jax.nn.sigmoid(sk1)[:, None, None]
  resw = (1.0 - sig0)[node_type][:, None]
  zero_resw = jnp.zeros((n, 1), jnp.float32)

  nt8 = (node_type[:, None] == jnp.arange(8, dtype=jnp.int32)[None, :]
         ).astype(jnp.float32)

  kidx = edge_type * n + src

  def chunked(ch):
    blk = NW * 8 * ch
    e_pad = ((e + blk - 1) // blk) * blk
    k2d = jnp.pad(kidx, [(0, e_pad - e)]).reshape(-1, ch)
    d2d = jnp.pad(dst, [(0, e_pad - e)],
                  constant_values=n_pad - 1).reshape(-1, ch)
    return k2d, d2d, e_pad // ch

  k2d0, d2d0, n_chunks0 = chunked(32)
  k2d1, d2d1, n_chunks1 = chunked(128)

  table0 = _proj(x, nt8, w9_0, rows=400)
  ek0 = _make_edge_kernel(n, e, n_chunks0, h0, hid, h0 * hid + 16, 32,
                          r_num * n)
  acc0 = ek0(table0, k2d0, d2d0)
  h = _comb(acc0, x, nt8, resw, wa0s, h0, hid, rows=400, resid=True)

  table1 = _proj(h, nt8, w9_1, rows=400)
  ek1 = _make_edge_kernel(n, e, n_chunks1, 1, out_d, out_d + 16, 128,
                          r_num * n)
  acc1 = ek1(table1, k2d1, d2d1)
  return _comb(acc1, h, nt8, zero_resw, wa1s, 1, out_d, rows=400, resid=False)


# speedup vs baseline: 33.6162x; 1.0078x over previous
import functools
import math

import jax
import jax.numpy as jnp
from jax import lax
from jax.experimental import pallas as pl
from jax.experimental.pallas import tpu as pltpu
from jax.experimental.pallas import tpu_sc as plsc

NC = 2
NS = 16
NW = NC * NS
CH = 128

_HIGH = jax.lax.Precision.HIGHEST


def _dot(a, b):
  return jnp.dot(a, b, preferred_element_type=jnp.float32, precision=_HIGH)


def _acc_rows(n_nodes):
  rpt = (-(-n_nodes // NS) + 7) // 8 * 8
  return rpt, rpt * NS


def _proj_body(x_ref, nt8_ref, w_ref, o_ref):
  nj, T = w_ref.shape[0], w_ref.shape[1]
  x = x_ref[...]
  xm = [x * nt8_ref[:, t:t + 1] for t in range(T)]
  for j in range(nj):
    o = _dot(xm[0], w_ref[j, 0])
    for t in range(1, T):
      o = o + _dot(xm[t], w_ref[j, t])
    o_ref[j] = o


def _proj(x, nt8, w9, rows):
  n, d_in = x.shape
  nj, t_, _, d_out = w9.shape
  grid_i = n // rows
  out = pl.pallas_call(
      _proj_body,
      grid=(grid_i,),
      in_specs=[
          pl.BlockSpec((rows, d_in), lambda i: (i, 0)),
          pl.BlockSpec((rows, 8), lambda i: (i, 0)),
          pl.BlockSpec((nj, t_, d_in, d_out), lambda i: (0, 0, 0, 0)),
      ],
      out_specs=pl.BlockSpec((nj, rows, d_out), lambda i: (0, i, 0)),
      out_shape=jax.ShapeDtypeStruct((nj, n, d_out), jnp.float32),
  )(x, nt8, w9)
  return out.reshape(nj * n, d_out)


def _make_edge_kernel(n_nodes, n_edges, n_chunks, heads, d, rw, ch, rn):
  hd = heads * d
  cpt = n_chunks // NW
  rpt, n_pad = _acc_rows(n_nodes)
  assert n_chunks % NW == 0 and cpt % 2 == 0
  nvec = d // 16
  voff = rn
  qoff = 2 * rn
  nv = -(-n_edges // ch)

  def body(table_hbm, kidx_hbm, dst_hbm, out_hbm,
           kidx_v, dst_v, vidx_r, qidx_r, kbuf, qbuf, vbuf, mbuf,
           acc, gsem0, gsem1):
    c = lax.axis_index("c")
    s = lax.axis_index("s")
    wid = s * NC + c
    t_ch0 = wid * cpt
    gsem = (gsem0, gsem1)

    for i in range(8):
      for j in range(rw // 16):
        mbuf[i, pl.ds(j * 16, 16)] = jnp.zeros((16,), jnp.float32)

    def zc(i, _):
      pltpu.sync_copy(mbuf.at[pl.ds(0, 8)], acc.at[pl.ds(s * rpt + i * 8, 8)])
      return 0
    lax.fori_loop(0, rpt // 8, zc, 0)
    plsc.subcore_barrier()

    pltpu.sync_copy(kidx_hbm.at[pl.ds(t_ch0, cpt)], kidx_v)
    pltpu.sync_copy(dst_hbm.at[pl.ds(t_ch0, cpt)], dst_v)

    lane = lax.iota(jnp.int32, 16)

    def chunk_ok(ci):
      return jnp.logical_and(t_ch0 + ci < nv, ci < cpt)

    def start(ci, par):
      for j in range(ch // 16):
        sl = pl.ds(j * 16, 16)
        vidx_r[par, sl] = kidx_v[ci, sl] + voff
        qidx_r[par, sl] = dst_v[ci, sl] + qoff
      pltpu.async_copy(table_hbm.at[kidx_v.at[ci]], kbuf.at[par], gsem[par])
      pltpu.async_copy(table_hbm.at[qidx_r.at[par]], qbuf.at[par], gsem[par])
      pltpu.async_copy(table_hbm.at[vidx_r.at[par]], vbuf.at[par], gsem[par])

    def finish(ci, par):
      dummy = table_hbm.at[pl.ds(0, ch)]
      pltpu.make_async_copy(dummy, kbuf.at[par], gsem[par]).wait()
      pltpu.make_async_copy(dummy, qbuf.at[par], gsem[par]).wait()
      pltpu.make_async_copy(dummy, vbuf.at[par], gsem[par]).wait()

      def one_edge(e):
        traws = []
        for h in range(heads):
          p = kbuf[par, e, pl.ds(h * d, 16)] * qbuf[par, e, pl.ds(h * d, 16)]
          for j in range(1, nvec):
            off = h * d + j * 16
            p = p + kbuf[par, e, pl.ds(off, 16)] * qbuf[par, e, pl.ds(off, 16)]
          traws.append(jnp.sum(p))
        denrow = jnp.zeros((16,), jnp.float32)
        for h in range(heads):
          exv = jnp.exp(jnp.full((16,), traws[h], jnp.float32))
          for j in range(nvec):
            off = h * d + j * 16
            mbuf[e, pl.ds(off, 16)] = exv * vbuf[par, e, pl.ds(off, 16)]
          denrow = denrow + jnp.where(lane == h, exv, 0.0)
        mbuf[e, pl.ds(hd, 16)] = denrow

      def edge_body(e2, _):
        for u in range(4):
          one_edge(e2 * 4 + u)
        return 0

      lax.fori_loop(0, ch // 4, edge_body, 0)
      pltpu.sync_copy(mbuf, acc.at[dst_v.at[ci]], add=True)

    @pl.when(chunk_ok(0))
    def _():
      start(0, 0)

    def pair_body(i2, _):
      ci0 = i2 * 2
      ci1 = ci0 + 1

      @pl.when(chunk_ok(ci1))
      def _():
        start(ci1, 1)

      @pl.when(chunk_ok(ci0))
      def _():
        finish(ci0, 0)

      @pl.when(chunk_ok(ci0 + 2))
      def _():
        start(ci0 + 2, 0)

      @pl.when(chunk_ok(ci1))
      def _():
        finish(ci1, 1)
      return 0

    lax.fori_loop(0, cpt // 2, pair_body, 0)
    plsc.subcore_barrier()
    pltpu.sync_copy(acc.at[pl.ds(s * rpt, rpt)],
                    out_hbm.at[c, pl.ds(s * rpt, rpt)])

  mesh = plsc.VectorSubcoreMesh(core_axis_name="c", subcore_axis_name="s",
                                num_cores=NC, num_subcores=NS)
  return pl.kernel(
      body,
      out_type=jax.ShapeDtypeStruct((NC, n_pad, rw), jnp.float32),
      mesh=mesh,
      compiler_params=pltpu.CompilerParams(needs_layout_passes=False,
                                           use_tc_tiling_on_sc=False),
      scratch_types=[
          pltpu.VMEM((cpt, ch), jnp.int32),
          pltpu.VMEM((cpt, ch), jnp.int32),
          pltpu.VMEM((2, ch), jnp.int32),
          pltpu.VMEM((2, ch), jnp.int32),
          pltpu.VMEM((2, ch, hd), jnp.float32),
          pltpu.VMEM((2, ch, hd), jnp.float32),
          pltpu.VMEM((2, ch, hd), jnp.float32),
          pltpu.VMEM((ch, rw), jnp.float32),
          pltpu.VMEM_SHARED((n_pad, rw), jnp.float32),
          pltpu.SemaphoreType.DMA,
          pltpu.SemaphoreType.DMA,
      ],
  )


def _comb_body(acc_ref, x_ref, nt8_ref, resw_ref, wa_ref, o_ref, heads, d,
               resid):
  T = wa_ref.shape[0]
  hd = heads * d
  s = acc_ref[0] + acc_ref[1]
  num = s[:, :hd]
  den = s[:, hd:hd + heads]
  inv = 1.0 / (den + 1e-9)
  if heads > 1:
    rowi = lax.broadcasted_iota(jnp.int32, (heads, hd), 0)
    coli = lax.broadcasted_iota(jnp.int32, (heads, hd), 1)
    sel = (coli // d == rowi).astype(jnp.float32)
    invwide = _dot(inv, sel)
  else:
    invwide = inv
  h_att = num * invwide
  o = jnp.zeros(o_ref.shape, jnp.float32)
  for t in range(T):
    hm = h_att * nt8_ref[:, t:t + 1]
    o = o + _dot(hm, wa_ref[t])
  if resid:
    o = o + x_ref[...] * resw_ref[...]
  o_ref[...] = o


def _comb(acc, x, nt8, resw, wa, heads, d, rows, resid):
  n = x.shape[0]
  rw = acc.shape[2]
  t_, _, d_out = wa.shape
  grid = n // rows
  body = functools.partial(_comb_body, heads=heads, d=d, resid=resid)
  return pl.pallas_call(
      body,
      grid=(grid,),
      in_specs=[
          pl.BlockSpec((NC, rows, rw), lambda i: (0, i, 0)),
          pl.BlockSpec((rows, x.shape[1]), lambda i: (i, 0)),
          pl.BlockSpec((rows, 8), lambda i: (i, 0)),
          pl.BlockSpec((rows, 1), lambda i: (i, 0)),
          pl.BlockSpec((t_, wa.shape[1], d_out), lambda i: (0, 0, 0)),
      ],
      out_specs=pl.BlockSpec((rows, d_out), lambda i: (i, 0)),
      out_shape=jax.ShapeDtypeStruct((n, d_out), jnp.float32),
  )(acc, x, nt8, resw, wa)


def kernel(x, edge_index, node_type, edge_type, Wk0, Wq0, Wv0, ra0, rm0, rp0,
           Wa0, sk0, Wk1, Wq1, Wv1, ra1, rm1, rp1, Wa1, sk1):
  n, d_in = x.shape
  e = edge_index.shape[1]
  t_num = Wk0.shape[0]
  r_num = ra0.shape[0]
  h0, hid = ra0.shape[1], ra0.shape[2]
  out_d = ra1.shape[2]
  _, n_pad = _acc_rows(n)

  src = edge_index[0]
  dst = edge_index[1]

  def fused_tables(wk, wv, wq, ra, rm, rp, heads, dh):
    din = wk.shape[1]
    wkr = wk.reshape(t_num, din, heads, dh)
    wvr = wv.reshape(t_num, din, heads, dh)
    scale = 1.0 / math.sqrt(dh)
    wkra = jnp.einsum("tihd,rhdf,rh->rtihf", wkr, ra, rp).reshape(
        r_num, t_num, din, heads * dh)
    wvrm = jnp.einsum("tihd,rhdf->rtihf", wvr, rm).reshape(
        r_num, t_num, din, heads * dh)
    return jnp.concatenate(
        [wkra, wvrm, (wq * scale)[None]], axis=0)

  w9_0 = fused_tables(Wk0, Wv0, Wq0, ra0, rm0, rp0, h0, hid)
  w9_1 = fused_tables(Wk1, Wv1, Wq1, ra1, rm1, rp1, 1, out_d)
  sig0 = jax.nn.sigmoid(sk0)
  wa0s = Wa0 * sig0[:, None, None]
  wa1s = Wa1 * jax.nn.sigmoid(sk1)[:, None, None]
  resw = (1.0 - sig0)[node_type][:, None]
  zero_resw = jnp.zeros((n, 1), jnp.float32)

  nt8 = (node_type[:, None] == jnp.arange(8, dtype=jnp.int32)[None, :]
         ).astype(jnp.float32)

  kidx = edge_type * n + src

  def chunked(ch):
    blk = NW * 8 * ch
    e_pad = ((e + blk - 1) // blk) * blk
    k2d = jnp.pad(kidx, [(0, e_pad - e)]).reshape(-1, ch)
    d2d = jnp.pad(dst, [(0, e_pad - e)],
                  constant_values=n_pad - 1).reshape(-1, ch)
    return k2d, d2d, e_pad // ch

  k2d0, d2d0, n_chunks0 = chunked(32)
  k2d1, d2d1, n_chunks1 = chunked(128)

  table0 = _proj(x, nt8, w9_0, rows=400)
  ek0 = _make_edge_kernel(n, e, n_chunks0, h0, hid, h0 * hid + 16, 32,
                          r_num * n)
  acc0 = ek0(table0, k2d0, d2d0)
  h = _comb(acc0, x, nt8, resw, wa0s, h0, hid, rows=400, resid=True)

  table1 = _proj(h, nt8, w9_1, rows=400)
  ek1 = _make_edge_kernel(n, e, n_chunks1, 1, out_d, out_d + 16, 128,
                          r_num * n)
  acc1 = ek1(table1, k2d1, d2d1)
  return _comb(acc1, h, nt8, zero_resw, wa1s, 1, out_d, rows=400, resid=False)


# speedup vs baseline: 42.4058x; 1.2615x over previous
import functools
import math

import jax
import jax.numpy as jnp
from jax import lax
from jax.experimental import pallas as pl
from jax.experimental.pallas import tpu as pltpu
from jax.experimental.pallas import tpu_sc as plsc

NC = 2
NS = 16
NW = NC * NS
CH = 128

_HIGH = jax.lax.Precision.DEFAULT


def _dot(a, b):
  return jnp.dot(a, b, preferred_element_type=jnp.float32, precision=_HIGH)


def _acc_rows(n_nodes):
  rpt = (-(-n_nodes // NS) + 7) // 8 * 8
  return rpt, rpt * NS


def _proj_body(x_ref, nt8_ref, w_ref, o_ref):
  nj, T = w_ref.shape[0], w_ref.shape[1]
  x = x_ref[...]
  xm = [x * nt8_ref[:, t:t + 1] for t in range(T)]
  for j in range(nj):
    o = _dot(xm[0], w_ref[j, 0])
    for t in range(1, T):
      o = o + _dot(xm[t], w_ref[j, t])
    o_ref[j] = o


def _proj(x, nt8, w9, rows):
  n, d_in = x.shape
  nj, t_, _, d_out = w9.shape
  grid_i = n // rows
  out = pl.pallas_call(
      _proj_body,
      grid=(grid_i,),
      in_specs=[
          pl.BlockSpec((rows, d_in), lambda i: (i, 0)),
          pl.BlockSpec((rows, 8), lambda i: (i, 0)),
          pl.BlockSpec((nj, t_, d_in, d_out), lambda i: (0, 0, 0, 0)),
      ],
      out_specs=pl.BlockSpec((nj, rows, d_out), lambda i: (0, i, 0)),
      out_shape=jax.ShapeDtypeStruct((nj, n, d_out), jnp.float32),
  )(x, nt8, w9)
  return out.reshape(nj * n, d_out)


def _make_edge_kernel(n_nodes, n_edges, n_chunks, heads, d, rw, ch, rn):
  hd = heads * d
  cpt = n_chunks // NW
  rpt, n_pad = _acc_rows(n_nodes)
  assert n_chunks % NW == 0 and cpt % 2 == 0
  nvec = d // 16
  voff = rn
  qoff = 2 * rn
  nv = -(-n_edges // ch)

  def body(table_hbm, kidx_hbm, dst_hbm, out_hbm,
           kidx_v, dst_v, vidx_r, qidx_r, kbuf, qbuf, vbuf, mbuf,
           acc, gsem0, gsem1):
    c = lax.axis_index("c")
    s = lax.axis_index("s")
    wid = s * NC + c
    t_ch0 = wid * cpt
    gsem = (gsem0, gsem1)

    for i in range(8):
      for j in range(rw // 16):
        mbuf[i, pl.ds(j * 16, 16)] = jnp.zeros((16,), jnp.float32)

    def zc(i, _):
      pltpu.sync_copy(mbuf.at[pl.ds(0, 8)], acc.at[pl.ds(s * rpt + i * 8, 8)])
      return 0
    lax.fori_loop(0, rpt // 8, zc, 0)
    plsc.subcore_barrier()

    pltpu.sync_copy(kidx_hbm.at[pl.ds(t_ch0, cpt)], kidx_v)
    pltpu.sync_copy(dst_hbm.at[pl.ds(t_ch0, cpt)], dst_v)

    lane = lax.iota(jnp.int32, 16)

    def chunk_ok(ci):
      return jnp.logical_and(t_ch0 + ci < nv, ci < cpt)

    def start(ci, par):
      for j in range(ch // 16):
        sl = pl.ds(j * 16, 16)
        vidx_r[par, sl] = kidx_v[ci, sl] + voff
        qidx_r[par, sl] = dst_v[ci, sl] + qoff
      pltpu.async_copy(table_hbm.at[kidx_v.at[ci]], kbuf.at[par], gsem[par])
      pltpu.async_copy(table_hbm.at[qidx_r.at[par]], qbuf.at[par], gsem[par])
      pltpu.async_copy(table_hbm.at[vidx_r.at[par]], vbuf.at[par], gsem[par])

    def finish(ci, par):
      dummy = table_hbm.at[pl.ds(0, ch)]
      pltpu.make_async_copy(dummy, kbuf.at[par], gsem[par]).wait()
      pltpu.make_async_copy(dummy, qbuf.at[par], gsem[par]).wait()
      pltpu.make_async_copy(dummy, vbuf.at[par], gsem[par]).wait()

      def one_edge(e):
        traws = []
        for h in range(heads):
          p = kbuf[par, e, pl.ds(h * d, 16)] * qbuf[par, e, pl.ds(h * d, 16)]
          for j in range(1, nvec):
            off = h * d + j * 16
            p = p + kbuf[par, e, pl.ds(off, 16)] * qbuf[par, e, pl.ds(off, 16)]
          traws.append(jnp.sum(p))
        denrow = jnp.zeros((16,), jnp.float32)
        for h in range(heads):
          exv = jnp.exp(jnp.full((16,), traws[h], jnp.float32))
          for j in range(nvec):
            off = h * d + j * 16
            mbuf[e, pl.ds(off, 16)] = exv * vbuf[par, e, pl.ds(off, 16)]
          denrow = denrow + jnp.where(lane == h, exv, 0.0)
        mbuf[e, pl.ds(hd, 16)] = denrow

      def edge_body(e2, _):
        for u in range(4):
          one_edge(e2 * 4 + u)
        return 0

      lax.fori_loop(0, ch // 4, edge_body, 0)
      pltpu.sync_copy(mbuf, acc.at[dst_v.at[ci]], add=True)

    @pl.when(chunk_ok(0))
    def _():
      start(0, 0)

    def pair_body(i2, _):
      ci0 = i2 * 2
      ci1 = ci0 + 1

      @pl.when(chunk_ok(ci1))
      def _():
        start(ci1, 1)

      @pl.when(chunk_ok(ci0))
      def _():
        finish(ci0, 0)

      @pl.when(chunk_ok(ci0 + 2))
      def _():
        start(ci0 + 2, 0)

      @pl.when(chunk_ok(ci1))
      def _():
        finish(ci1, 1)
      return 0

    lax.fori_loop(0, cpt // 2, pair_body, 0)
    plsc.subcore_barrier()
    pltpu.sync_copy(acc.at[pl.ds(s * rpt, rpt)],
                    out_hbm.at[c, pl.ds(s * rpt, rpt)])

  mesh = plsc.VectorSubcoreMesh(core_axis_name="c", subcore_axis_name="s",
                                num_cores=NC, num_subcores=NS)
  return pl.kernel(
      body,
      out_type=jax.ShapeDtypeStruct((NC, n_pad, rw), jnp.float32),
      mesh=mesh,
      compiler_params=pltpu.CompilerParams(needs_layout_passes=False,
                                           use_tc_tiling_on_sc=False),
      scratch_types=[
          pltpu.VMEM((cpt, ch), jnp.int32),
          pltpu.VMEM((cpt, ch), jnp.int32),
          pltpu.VMEM((2, ch), jnp.int32),
          pltpu.VMEM((2, ch), jnp.int32),
          pltpu.VMEM((2, ch, hd), jnp.float32),
          pltpu.VMEM((2, ch, hd), jnp.float32),
          pltpu.VMEM((2, ch, hd), jnp.float32),
          pltpu.VMEM((ch, rw), jnp.float32),
          pltpu.VMEM_SHARED((n_pad, rw), jnp.float32),
          pltpu.SemaphoreType.DMA,
          pltpu.SemaphoreType.DMA,
      ],
  )


def _comb_body(acc_ref, x_ref, nt8_ref, resw_ref, wa_ref, o_ref, heads, d,
               resid):
  T = wa_ref.shape[0]
  hd = heads * d
  s = acc_ref[0] + acc_ref[1]
  num = s[:, :hd]
  den = s[:, hd:hd + heads]
  inv = 1.0 / (den + 1e-9)
  if heads > 1:
    rowi = lax.broadcasted_iota(jnp.int32, (heads, hd), 0)
    coli = lax.broadcasted_iota(jnp.int32, (heads, hd), 1)
    sel = (coli // d == rowi).astype(jnp.float32)
    invwide = _dot(inv, sel)
  else:
    invwide = inv
  h_att = num * invwide
  o = jnp.zeros(o_ref.shape, jnp.float32)
  for t in range(T):
    hm = h_att * nt8_ref[:, t:t + 1]
    o = o + _dot(hm, wa_ref[t])
  if resid:
    o = o + x_ref[...] * resw_ref[...]
  o_ref[...] = o


def _comb(acc, x, nt8, resw, wa, heads, d, rows, resid):
  n = x.shape[0]
  rw = acc.shape[2]
  t_, _, d_out = wa.shape
  grid = n // rows
  body = functools.partial(_comb_body, heads=heads, d=d, resid=resid)
  return pl.pallas_call(
      body,
      grid=(grid,),
      in_specs=[
          pl.BlockSpec((NC, rows, rw), lambda i: (0, i, 0)),
          pl.BlockSpec((rows, x.shape[1]), lambda i: (i, 0)),
          pl.BlockSpec((rows, 8), lambda i: (i, 0)),
          pl.BlockSpec((rows, 1), lambda i: (i, 0)),
          pl.BlockSpec((t_, wa.shape[1], d_out), lambda i: (0, 0, 0)),
      ],
      out_specs=pl.BlockSpec((rows, d_out), lambda i: (i, 0)),
      out_shape=jax.ShapeDtypeStruct((n, d_out), jnp.float32),
  )(acc, x, nt8, resw, wa)


def kernel(x, edge_index, node_type, edge_type, Wk0, Wq0, Wv0, ra0, rm0, rp0,
           Wa0, sk0, Wk1, Wq1, Wv1, ra1, rm1, rp1, Wa1, sk1):
  n, d_in = x.shape
  e = edge_index.shape[1]
  t_num = Wk0.shape[0]
  r_num = ra0.shape[0]
  h0, hid = ra0.shape[1], ra0.shape[2]
  out_d = ra1.shape[2]
  _, n_pad = _acc_rows(n)

  src = edge_index[0]
  dst = edge_index[1]

  def fused_tables(wk, wv, wq, ra, rm, rp, heads, dh):
    din = wk.shape[1]
    wkr = wk.reshape(t_num, din, heads, dh)
    wvr = wv.reshape(t_num, din, heads, dh)
    scale = 1.0 / math.sqrt(dh)
    wkra = jnp.einsum("tihd,rhdf,rh->rtihf", wkr, ra, rp).reshape(
        r_num, t_num, din, heads * dh)
    wvrm = jnp.einsum("tihd,rhdf->rtihf", wvr, rm).reshape(
        r_num, t_num, din, heads * dh)
    return jnp.concatenate(
        [wkra, wvrm, (wq * scale)[None]], axis=0)

  w9_0 = fused_tables(Wk0, Wv0, Wq0, ra0, rm0, rp0, h0, hid)
  w9_1 = fused_tables(Wk1, Wv1, Wq1, ra1, rm1, rp1, 1, out_d)
  sig0 = jax.nn.sigmoid(sk0)
  wa0s = Wa0 * sig0[:, None, None]
  wa1s = Wa1 * jax.nn.sigmoid(sk1)[:, None, None]
  resw = (1.0 - sig0)[node_type][:, None]
  zero_resw = jnp.zeros((n, 1), jnp.float32)

  nt8 = (node_type[:, None] == jnp.arange(8, dtype=jnp.int32)[None, :]
         ).astype(jnp.float32)

  kidx = edge_type * n + src

  def chunked(ch):
    blk = NW * 8 * ch
    e_pad = ((e + blk - 1) // blk) * blk
    k2d = jnp.pad(kidx, [(0, e_pad - e)]).reshape(-1, ch)
    d2d = jnp.pad(dst, [(0, e_pad - e)],
                  constant_values=n_pad - 1).reshape(-1, ch)
    return k2d, d2d, e_pad // ch

  k2d0, d2d0, n_chunks0 = chunked(32)
  k2d1, d2d1, n_chunks1 = chunked(128)

  table0 = _proj(x, nt8, w9_0, rows=400)
  ek0 = _make_edge_kernel(n, e, n_chunks0, h0, hid, h0 * hid + 16, 32,
                          r_num * n)
  acc0 = ek0(table0, k2d0, d2d0)
  h = _comb(acc0, x, nt8, resw, wa0s, h0, hid, rows=400, resid=True)

  table1 = _proj(h, nt8, w9_1, rows=400)
  ek1 = _make_edge_kernel(n, e, n_chunks1, 1, out_d, out_d + 16, 128,
                          r_num * n)
  acc1 = ek1(table1, k2d1, d2d1)
  return _comb(acc1, h, nt8, zero_resw, wa1s, 1, out_d, rows=400, resid=False)


# speedup vs baseline: 46.5816x; 1.0985x over previous
import functools
import math

import jax
import jax.numpy as jnp
from jax import lax
from jax.experimental import pallas as pl
from jax.experimental.pallas import tpu as pltpu
from jax.experimental.pallas import tpu_sc as plsc

NC = 2
NS = 16
NW = NC * NS
CH = 128

_HIGH = jax.lax.Precision.DEFAULT


def _dot(a, b):
  return jnp.dot(a, b, preferred_element_type=jnp.float32, precision=_HIGH)


def _acc_rows(n_nodes):
  rpt = (-(-n_nodes // NS) + 7) // 8 * 8
  return rpt, rpt * NS


def _proj_body(x_ref, nt8_ref, w_ref, o_ref):
  nj, T = w_ref.shape[0], w_ref.shape[1]
  x = x_ref[...]
  xm = [x * nt8_ref[:, t:t + 1] for t in range(T)]
  for j in range(nj):
    o = _dot(xm[0], w_ref[j, 0])
    for t in range(1, T):
      o = o + _dot(xm[t], w_ref[j, t])
    o_ref[j] = o.astype(o_ref.dtype)


def _proj(x, nt8, w9, rows, out_dtype=jnp.float32):
  n, d_in = x.shape
  nj, t_, _, d_out = w9.shape
  grid_i = n // rows
  out = pl.pallas_call(
      _proj_body,
      grid=(grid_i,),
      in_specs=[
          pl.BlockSpec((rows, d_in), lambda i: (i, 0)),
          pl.BlockSpec((rows, 8), lambda i: (i, 0)),
          pl.BlockSpec((nj, t_, d_in, d_out), lambda i: (0, 0, 0, 0)),
      ],
      out_specs=pl.BlockSpec((nj, rows, d_out), lambda i: (0, i, 0)),
      out_shape=jax.ShapeDtypeStruct((nj, n, d_out), out_dtype),
  )(x, nt8, w9)
  return out.reshape(nj * n, d_out)


def _make_edge_kernel(n_nodes, n_edges, n_chunks, heads, d, rw, ch, rn,
                      bf16_table=False):
  hd = heads * d
  cpt = n_chunks // NW
  rpt, n_pad = _acc_rows(n_nodes)
  assert n_chunks % NW == 0 and cpt % 2 == 0
  nvec = d // 16
  voff = rn
  qoff = 2 * rn
  nv = -(-n_edges // ch)
  tdt = jnp.bfloat16 if bf16_table else jnp.float32

  def body(table_hbm, kidx_hbm, dst_hbm, out_hbm,
           kidx_v, dst_v, vidx_r, qidx_r, kbuf, qbuf, vbuf, mbuf,
           acc, gsem0, gsem1):
    c = lax.axis_index("c")
    s = lax.axis_index("s")
    wid = s * NC + c
    t_ch0 = wid * cpt
    gsem = (gsem0, gsem1)

    for i in range(8):
      for j in range(rw // 16):
        mbuf[i, pl.ds(j * 16, 16)] = jnp.zeros((16,), jnp.float32)

    def zc(i, _):
      pltpu.sync_copy(mbuf.at[pl.ds(0, 8)], acc.at[pl.ds(s * rpt + i * 8, 8)])
      return 0
    lax.fori_loop(0, rpt // 8, zc, 0)
    plsc.subcore_barrier()

    pltpu.sync_copy(kidx_hbm.at[pl.ds(t_ch0, cpt)], kidx_v)
    pltpu.sync_copy(dst_hbm.at[pl.ds(t_ch0, cpt)], dst_v)

    lane = lax.iota(jnp.int32, 16)

    def chunk_ok(ci):
      return jnp.logical_and(t_ch0 + ci < nv, ci < cpt)

    def start(ci, par):
      for j in range(ch // 16):
        sl = pl.ds(j * 16, 16)
        vidx_r[par, sl] = kidx_v[ci, sl] + voff
        qidx_r[par, sl] = dst_v[ci, sl] + qoff
      pltpu.async_copy(table_hbm.at[kidx_v.at[ci]], kbuf.at[par], gsem[par])
      pltpu.async_copy(table_hbm.at[qidx_r.at[par]], qbuf.at[par], gsem[par])
      pltpu.async_copy(table_hbm.at[vidx_r.at[par]], vbuf.at[par], gsem[par])

    def finish(ci, par):
      dummy = table_hbm.at[pl.ds(0, ch)]
      pltpu.make_async_copy(dummy, kbuf.at[par], gsem[par]).wait()
      pltpu.make_async_copy(dummy, qbuf.at[par], gsem[par]).wait()
      pltpu.make_async_copy(dummy, vbuf.at[par], gsem[par]).wait()

      def one_edge(e):
        traws = []
        vparts = []
        for h in range(heads):
          if bf16_table:
            ka, kb = plsc.unpack(kbuf[par, e, pl.ds(h * d, d)],
                                 format=plsc.PackFormat.INTERLEAVED)
            qa, qb = plsc.unpack(qbuf[par, e, pl.ds(h * d, d)],
                                 format=plsc.PackFormat.INTERLEAVED)
            va, vb = plsc.unpack(vbuf[par, e, pl.ds(h * d, d)],
                                 format=plsc.PackFormat.INTERLEAVED)
            vparts.append((va, vb))
            p = ka * qa + kb * qb
          else:
            p = (kbuf[par, e, pl.ds(h * d, 16)] *
                 qbuf[par, e, pl.ds(h * d, 16)])
            for j in range(1, nvec):
              off = h * d + j * 16
              p = p + (kbuf[par, e, pl.ds(off, 16)] *
                       qbuf[par, e, pl.ds(off, 16)])
          traws.append(jnp.sum(p))
        denrow = jnp.zeros((16,), jnp.float32)
        for h in range(heads):
          exv = jnp.exp(jnp.full((16,), traws[h], jnp.float32))
          if bf16_table:
            va, vb = vparts[h]
            mbuf[e, pl.ds(h * d, 16)] = exv * va
            mbuf[e, pl.ds(h * d + 16, 16)] = exv * vb
          else:
            for j in range(nvec):
              off = h * d + j * 16
              mbuf[e, pl.ds(off, 16)] = exv * vbuf[par, e, pl.ds(off, 16)]
          denrow = denrow + jnp.where(lane == h, exv, 0.0)
        mbuf[e, pl.ds(hd, 16)] = denrow

      def edge_body(e2, _):
        for u in range(4):
          one_edge(e2 * 4 + u)
        return 0

      lax.fori_loop(0, ch // 4, edge_body, 0)
      pltpu.sync_copy(mbuf, acc.at[dst_v.at[ci]], add=True)

    @pl.when(chunk_ok(0))
    def _():
      start(0, 0)

    def pair_body(i2, _):
      ci0 = i2 * 2
      ci1 = ci0 + 1

      @pl.when(chunk_ok(ci1))
      def _():
        start(ci1, 1)

      @pl.when(chunk_ok(ci0))
      def _():
        finish(ci0, 0)

      @pl.when(chunk_ok(ci0 + 2))
      def _():
        start(ci0 + 2, 0)

      @pl.when(chunk_ok(ci1))
      def _():
        finish(ci1, 1)
      return 0

    lax.fori_loop(0, cpt // 2, pair_body, 0)
    plsc.subcore_barrier()
    pltpu.sync_copy(acc.at[pl.ds(s * rpt, rpt)],
                    out_hbm.at[c, pl.ds(s * rpt, rpt)])

  mesh = plsc.VectorSubcoreMesh(core_axis_name="c", subcore_axis_name="s",
                                num_cores=NC, num_subcores=NS)
  return pl.kernel(
      body,
      out_type=jax.ShapeDtypeStruct((NC, n_pad, rw), jnp.float32),
      mesh=mesh,
      compiler_params=pltpu.CompilerParams(needs_layout_passes=False,
                                           use_tc_tiling_on_sc=False),
      scratch_types=[
          pltpu.VMEM((cpt, ch), jnp.int32),
          pltpu.VMEM((cpt, ch), jnp.int32),
          pltpu.VMEM((2, ch), jnp.int32),
          pltpu.VMEM((2, ch), jnp.int32),
          pltpu.VMEM((2, ch, hd), tdt),
          pltpu.VMEM((2, ch, hd), tdt),
          pltpu.VMEM((2, ch, hd), tdt),
          pltpu.VMEM((ch, rw), jnp.float32),
          pltpu.VMEM_SHARED((n_pad, rw), jnp.float32),
          pltpu.SemaphoreType.DMA,
          pltpu.SemaphoreType.DMA,
      ],
  )


def _comb_body(acc_ref, x_ref, nt8_ref, resw_ref, wa_ref, o_ref, heads, d,
               resid):
  T = wa_ref.shape[0]
  hd = heads * d
  s = acc_ref[0] + acc_ref[1]
  num = s[:, :hd]
  den = s[:, hd:hd + heads]
  inv = 1.0 / (den + 1e-9)
  if heads > 1:
    rowi = lax.broadcasted_iota(jnp.int32, (heads, hd), 0)
    coli = lax.broadcasted_iota(jnp.int32, (heads, hd), 1)
    sel = (coli // d == rowi).astype(jnp.float32)
    invwide = _dot(inv, sel)
  else:
    invwide = inv
  h_att = num * invwide
  o = jnp.zeros(o_ref.shape, jnp.float32)
  for t in range(T):
    hm = h_att * nt8_ref[:, t:t + 1]
    o = o + _dot(hm, wa_ref[t])
  if resid:
    o = o + x_ref[...] * resw_ref[...]
  o_ref[...] = o


def _comb(acc, x, nt8, resw, wa, heads, d, rows, resid):
  n = x.shape[0]
  rw = acc.shape[2]
  t_, _, d_out = wa.shape
  grid = n // rows
  body = functools.partial(_comb_body, heads=heads, d=d, resid=resid)
  return pl.pallas_call(
      body,
      grid=(grid,),
      in_specs=[
          pl.BlockSpec((NC, rows, rw), lambda i: (0, i, 0)),
          pl.BlockSpec((rows, x.shape[1]), lambda i: (i, 0)),
          pl.BlockSpec((rows, 8), lambda i: (i, 0)),
          pl.BlockSpec((rows, 1), lambda i: (i, 0)),
          pl.BlockSpec((t_, wa.shape[1], d_out), lambda i: (0, 0, 0)),
      ],
      out_specs=pl.BlockSpec((rows, d_out), lambda i: (i, 0)),
      out_shape=jax.ShapeDtypeStruct((n, d_out), jnp.float32),
  )(acc, x, nt8, resw, wa)


def kernel(x, edge_index, node_type, edge_type, Wk0, Wq0, Wv0, ra0, rm0, rp0,
           Wa0, sk0, Wk1, Wq1, Wv1, ra1, rm1, rp1, Wa1, sk1):
  n, d_in = x.shape
  e = edge_index.shape[1]
  t_num = Wk0.shape[0]
  r_num = ra0.shape[0]
  h0, hid = ra0.shape[1], ra0.shape[2]
  out_d = ra1.shape[2]
  _, n_pad = _acc_rows(n)

  src = edge_index[0]
  dst = edge_index[1]

  def fused_tables(wk, wv, wq, ra, rm, rp, heads, dh):
    din = wk.shape[1]
    wkr = wk.reshape(t_num, din, heads, dh)
    wvr = wv.reshape(t_num, din, heads, dh)
    scale = 1.0 / math.sqrt(dh)
    wkra = jnp.einsum("tihd,rhdf,rh->rtihf", wkr, ra, rp).reshape(
        r_num, t_num, din, heads * dh)
    wvrm = jnp.einsum("tihd,rhdf->rtihf", wvr, rm).reshape(
        r_num, t_num, din, heads * dh)
    return jnp.concatenate(
        [wkra, wvrm, (wq * scale)[None]], axis=0)

  w9_0 = fused_tables(Wk0, Wv0, Wq0, ra0, rm0, rp0, h0, hid)
  w9_1 = fused_tables(Wk1, Wv1, Wq1, ra1, rm1, rp1, 1, out_d)
  perm = jnp.arange(h0 * hid).reshape(h0, hid // 2, 2).transpose(0, 2, 1)
  perm = perm.reshape(-1)
  sig0 = jax.nn.sigmoid(sk0)
  wa0s = (Wa0 * sig0[:, None, None])[:, perm, :]
  wa1s = Wa1 * jax.nn.sigmoid(sk1)[:, None, None]
  resw = (1.0 - sig0)[node_type][:, None]
  zero_resw = jnp.zeros((n, 1), jnp.float32)

  nt8 = (node_type[:, None] == jnp.arange(8, dtype=jnp.int32)[None, :]
         ).astype(jnp.float32)

  kidx = edge_type * n + src

  def chunked(ch):
    blk = NW * 8 * ch
    e_pad = ((e + blk - 1) // blk) * blk
    k2d = jnp.pad(kidx, [(0, e_pad - e)]).reshape(-1, ch)
    d2d = jnp.pad(dst, [(0, e_pad - e)],
                  constant_values=n_pad - 1).reshape(-1, ch)
    return k2d, d2d, e_pad // ch

  k2d0, d2d0, n_chunks0 = chunked(32)
  k2d1, d2d1, n_chunks1 = chunked(128)

  table0 = _proj(x, nt8, w9_0, rows=400, out_dtype=jnp.bfloat16)
  ek0 = _make_edge_kernel(n, e, n_chunks0, h0, hid, h0 * hid + 16, 32,
                          r_num * n, bf16_table=True)
  acc0 = ek0(table0, k2d0, d2d0)
  h = _comb(acc0, x, nt8, resw, wa0s, h0, hid, rows=400, resid=True)

  table1 = _proj(h, nt8, w9_1, rows=400)
  ek1 = _make_edge_kernel(n, e, n_chunks1, 1, out_d, out_d + 16, 128,
                          r_num * n)
  acc1 = ek1(table1, k2d1, d2d1)
  return _comb(acc1, h, nt8, zero_resw, wa1s, 1, out_d, rows=400, resid=False)


# speedup vs baseline: 49.9892x; 1.0732x over previous
import functools
import math

import jax
import jax.numpy as jnp
from jax import lax
from jax.experimental import pallas as pl
from jax.experimental.pallas import tpu as pltpu
from jax.experimental.pallas import tpu_sc as plsc

NC = 2
NS = 16
NW = NC * NS
CH = 128

_HIGH = jax.lax.Precision.DEFAULT


def _dot(a, b):
  return jnp.dot(a, b, preferred_element_type=jnp.float32, precision=_HIGH)


def _acc_rows(n_nodes):
  rpt = (-(-n_nodes // NS) + 7) // 8 * 8
  return rpt, rpt * NS


def _proj_body(x_ref, nt8_ref, w_ref, o_ref):
  nj, T = w_ref.shape[0], w_ref.shape[1]
  x = x_ref[...]
  xm = [x * nt8_ref[:, t:t + 1] for t in range(T)]
  for j in range(nj):
    o = _dot(xm[0], w_ref[j, 0])
    for t in range(1, T):
      o = o + _dot(xm[t], w_ref[j, t])
    o_ref[j] = o.astype(o_ref.dtype)


def _proj(x, nt8, w9, rows, out_dtype=jnp.float32):
  n, d_in = x.shape
  nj, t_, _, d_out = w9.shape
  grid_i = n // rows
  out = pl.pallas_call(
      _proj_body,
      grid=(grid_i,),
      in_specs=[
          pl.BlockSpec((rows, d_in), lambda i: (i, 0)),
          pl.BlockSpec((rows, 8), lambda i: (i, 0)),
          pl.BlockSpec((nj, t_, d_in, d_out), lambda i: (0, 0, 0, 0)),
      ],
      out_specs=pl.BlockSpec((nj, rows, d_out), lambda i: (0, i, 0)),
      out_shape=jax.ShapeDtypeStruct((nj, n, d_out), out_dtype),
  )(x, nt8, w9)
  return out.reshape(nj * n, d_out)


def _make_edge_kernel(n_nodes, n_edges, n_chunks, heads, d, rw, ch, rn,
                      bf16_table=False):
  hd = heads * d
  cpt = n_chunks // NW
  rpt, n_pad = _acc_rows(n_nodes)
  assert n_chunks % NW == 0 and cpt % 2 == 0
  nvec = d // 16
  voff = rn
  qoff = 2 * rn
  nv = -(-n_edges // ch)
  tdt = jnp.bfloat16 if bf16_table else jnp.float32

  def body(table_hbm, kidx_hbm, dst_hbm, out_hbm,
           kidx_v, dst_v, vidx_r, qidx_r, kbuf, qbuf, vbuf, mbuf,
           acc, gsem0, gsem1, ssem0, ssem1):
    c = lax.axis_index("c")
    s = lax.axis_index("s")
    wid = s * NC + c
    t_ch0 = wid * cpt
    gsem = (gsem0, gsem1)
    ssem = (ssem0, ssem1)

    for i in range(8):
      for j in range(rw // 16):
        mbuf[0, i, pl.ds(j * 16, 16)] = jnp.zeros((16,), jnp.float32)

    def zc(i, _):
      pltpu.sync_copy(mbuf.at[0, pl.ds(0, 8)],
                      acc.at[pl.ds(s * rpt + i * 8, 8)])
      return 0
    lax.fori_loop(0, rpt // 8, zc, 0)
    plsc.subcore_barrier()

    pltpu.sync_copy(kidx_hbm.at[pl.ds(t_ch0, cpt)], kidx_v)
    pltpu.sync_copy(dst_hbm.at[pl.ds(t_ch0, cpt)], dst_v)

    lane = lax.iota(jnp.int32, 16)

    def chunk_ok(ci):
      return jnp.logical_and(t_ch0 + ci < nv, ci < cpt)

    def start(ci, par):
      for j in range(ch // 16):
        sl = pl.ds(j * 16, 16)
        vidx_r[par, sl] = kidx_v[ci, sl] + voff
        qidx_r[par, sl] = dst_v[ci, sl] + qoff
      pltpu.async_copy(table_hbm.at[kidx_v.at[ci]], kbuf.at[par], gsem[par])
      pltpu.async_copy(table_hbm.at[qidx_r.at[par]], qbuf.at[par], gsem[par])
      pltpu.async_copy(table_hbm.at[vidx_r.at[par]], vbuf.at[par], gsem[par])

    def finish(ci, par):
      dummy = table_hbm.at[pl.ds(0, ch)]
      pltpu.make_async_copy(dummy, kbuf.at[par], gsem[par]).wait()
      pltpu.make_async_copy(dummy, qbuf.at[par], gsem[par]).wait()
      pltpu.make_async_copy(dummy, vbuf.at[par], gsem[par]).wait()

      @pl.when(ci >= 2)
      def _():
        pltpu.make_async_copy(mbuf.at[par], acc.at[dst_v.at[ci - 2]],
                              ssem[par]).wait()

      def one_edge(e):
        traws = []
        vparts = []
        for h in range(heads):
          if bf16_table:
            ka, kb = plsc.unpack(kbuf[par, e, pl.ds(h * d, d)],
                                 format=plsc.PackFormat.INTERLEAVED)
            qa, qb = plsc.unpack(qbuf[par, e, pl.ds(h * d, d)],
                                 format=plsc.PackFormat.INTERLEAVED)
            va, vb = plsc.unpack(vbuf[par, e, pl.ds(h * d, d)],
                                 format=plsc.PackFormat.INTERLEAVED)
            vparts.append((va, vb))
            p = ka * qa + kb * qb
          else:
            p = (kbuf[par, e, pl.ds(h * d, 16)] *
                 qbuf[par, e, pl.ds(h * d, 16)])
            for j in range(1, nvec):
              off = h * d + j * 16
              p = p + (kbuf[par, e, pl.ds(off, 16)] *
                       qbuf[par, e, pl.ds(off, 16)])
          traws.append(jnp.sum(p))
        denrow = jnp.zeros((16,), jnp.float32)
        for h in range(heads):
          exv = jnp.exp(jnp.full((16,), traws[h], jnp.float32))
          if bf16_table:
            va, vb = vparts[h]
            mbuf[par, e, pl.ds(h * d, 16)] = exv * va
            mbuf[par, e, pl.ds(h * d + 16, 16)] = exv * vb
          else:
            for j in range(nvec):
              off = h * d + j * 16
              mbuf[par, e, pl.ds(off, 16)] = exv * vbuf[par, e, pl.ds(off, 16)]
          denrow = denrow + jnp.where(lane == h, exv, 0.0)
        mbuf[par, e, pl.ds(hd, 16)] = denrow

      def edge_body(e2, _):
        for u in range(4):
          one_edge(e2 * 4 + u)
        return 0

      lax.fori_loop(0, ch // 4, edge_body, 0)
      pltpu.async_copy(mbuf.at[par], acc.at[dst_v.at[ci]], ssem[par],
                       add=True)

    @pl.when(chunk_ok(0))
    def _():
      start(0, 0)

    def pair_body(i2, _):
      ci0 = i2 * 2
      ci1 = ci0 + 1

      @pl.when(chunk_ok(ci1))
      def _():
        start(ci1, 1)

      @pl.when(chunk_ok(ci0))
      def _():
        finish(ci0, 0)

      @pl.when(chunk_ok(ci0 + 2))
      def _():
        start(ci0 + 2, 0)

      @pl.when(chunk_ok(ci1))
      def _():
        finish(ci1, 1)
      return 0

    lax.fori_loop(0, cpt // 2, pair_body, 0)

    for par in range(2):
      @pl.when(chunk_ok(par))
      def _():
        pltpu.make_async_copy(mbuf.at[par], acc.at[dst_v.at[par]],
                              ssem[par]).wait()
    plsc.subcore_barrier()
    pltpu.sync_copy(acc.at[pl.ds(s * rpt, rpt)],
                    out_hbm.at[c, pl.ds(s * rpt, rpt)])

  mesh = plsc.VectorSubcoreMesh(core_axis_name="c", subcore_axis_name="s",
                                num_cores=NC, num_subcores=NS)
  return pl.kernel(
      body,
      out_type=jax.ShapeDtypeStruct((NC, n_pad, rw), jnp.float32),
      mesh=mesh,
      compiler_params=pltpu.CompilerParams(needs_layout_passes=False,
                                           use_tc_tiling_on_sc=False),
      scratch_types=[
          pltpu.VMEM((cpt, ch), jnp.int32),
          pltpu.VMEM((cpt, ch), jnp.int32),
          pltpu.VMEM((2, ch), jnp.int32),
          pltpu.VMEM((2, ch), jnp.int32),
          pltpu.VMEM((2, ch, hd), tdt),
          pltpu.VMEM((2, ch, hd), tdt),
          pltpu.VMEM((2, ch, hd), tdt),
          pltpu.VMEM((2, ch, rw), jnp.float32),
          pltpu.VMEM_SHARED((n_pad, rw), jnp.float32),
          pltpu.SemaphoreType.DMA,
          pltpu.SemaphoreType.DMA,
          pltpu.SemaphoreType.DMA,
          pltpu.SemaphoreType.DMA,
      ],
  )


def _comb_body(acc_ref, x_ref, nt8_ref, resw_ref, wa_ref, o_ref, heads, d,
               resid):
  T = wa_ref.shape[0]
  hd = heads * d
  s = acc_ref[0] + acc_ref[1]
  num = s[:, :hd]
  den = s[:, hd:hd + heads]
  inv = 1.0 / (den + 1e-9)
  if heads > 1:
    rowi = lax.broadcasted_iota(jnp.int32, (heads, hd), 0)
    coli = lax.broadcasted_iota(jnp.int32, (heads, hd), 1)
    sel = (coli // d == rowi).astype(jnp.float32)
    invwide = _dot(inv, sel)
  else:
    invwide = inv
  h_att = num * invwide
  o = jnp.zeros(o_ref.shape, jnp.float32)
  for t in range(T):
    hm = h_att * nt8_ref[:, t:t + 1]
    o = o + _dot(hm, wa_ref[t])
  if resid:
    o = o + x_ref[...] * resw_ref[...]
  o_ref[...] = o


def _comb(acc, x, nt8, resw, wa, heads, d, rows, resid):
  n = x.shape[0]
  rw = acc.shape[2]
  t_, _, d_out = wa.shape
  grid = n // rows
  body = functools.partial(_comb_body, heads=heads, d=d, resid=resid)
  return pl.pallas_call(
      body,
      grid=(grid,),
      in_specs=[
          pl.BlockSpec((NC, rows, rw), lambda i: (0, i, 0)),
          pl.BlockSpec((rows, x.shape[1]), lambda i: (i, 0)),
          pl.BlockSpec((rows, 8), lambda i: (i, 0)),
          pl.BlockSpec((rows, 1), lambda i: (i, 0)),
          pl.BlockSpec((t_, wa.shape[1], d_out), lambda i: (0, 0, 0)),
      ],
      out_specs=pl.BlockSpec((rows, d_out), lambda i: (i, 0)),
      out_shape=jax.ShapeDtypeStruct((n, d_out), jnp.float32),
  )(acc, x, nt8, resw, wa)


def kernel(x, edge_index, node_type, edge_type, Wk0, Wq0, Wv0, ra0, rm0, rp0,
           Wa0, sk0, Wk1, Wq1, Wv1, ra1, rm1, rp1, Wa1, sk1):
  n, d_in = x.shape
  e = edge_index.shape[1]
  t_num = Wk0.shape[0]
  r_num = ra0.shape[0]
  h0, hid = ra0.shape[1], ra0.shape[2]
  out_d = ra1.shape[2]
  _, n_pad = _acc_rows(n)

  src = edge_index[0]
  dst = edge_index[1]

  def fused_tables(wk, wv, wq, ra, rm, rp, heads, dh):
    din = wk.shape[1]
    wkr = wk.reshape(t_num, din, heads, dh)
    wvr = wv.reshape(t_num, din, heads, dh)
    scale = 1.0 / math.sqrt(dh)
    wkra = jnp.einsum("tihd,rhdf,rh->rtihf", wkr, ra, rp).reshape(
        r_num, t_num, din, heads * dh)
    wvrm = jnp.einsum("tihd,rhdf->rtihf", wvr, rm).reshape(
        r_num, t_num, din, heads * dh)
    return jnp.concatenate(
        [wkra, wvrm, (wq * scale)[None]], axis=0)

  w9_0 = fused_tables(Wk0, Wv0, Wq0, ra0, rm0, rp0, h0, hid)
  w9_1 = fused_tables(Wk1, Wv1, Wq1, ra1, rm1, rp1, 1, out_d)
  perm = jnp.arange(h0 * hid).reshape(h0, hid // 2, 2).transpose(0, 2, 1)
  perm = perm.reshape(-1)
  sig0 = jax.nn.sigmoid(sk0)
  wa0s = (Wa0 * sig0[:, None, None])[:, perm, :]
  wa1s = Wa1 * jax.nn.sigmoid(sk1)[:, None, None]
  resw = (1.0 - sig0)[node_type][:, None]
  zero_resw = jnp.zeros((n, 1), jnp.float32)

  nt8 = (node_type[:, None] == jnp.arange(8, dtype=jnp.int32)[None, :]
         ).astype(jnp.float32)

  kidx = edge_type * n + src

  def chunked(ch):
    blk = NW * 8 * ch
    e_pad = ((e + blk - 1) // blk) * blk
    k2d = jnp.pad(kidx, [(0, e_pad - e)]).reshape(-1, ch)
    d2d = jnp.pad(dst, [(0, e_pad - e)],
                  constant_values=n_pad - 1).reshape(-1, ch)
    return k2d, d2d, e_pad // ch

  k2d0, d2d0, n_chunks0 = chunked(32)
  k2d1, d2d1, n_chunks1 = chunked(128)

  table0 = _proj(x, nt8, w9_0, rows=400, out_dtype=jnp.bfloat16)
  ek0 = _make_edge_kernel(n, e, n_chunks0, h0, hid, h0 * hid + 16, 32,
                          r_num * n, bf16_table=True)
  acc0 = ek0(table0, k2d0, d2d0)
  h = _comb(acc0, x, nt8, resw, wa0s, h0, hid, rows=400, resid=True)

  table1 = _proj(h, nt8, w9_1, rows=400)
  ek1 = _make_edge_kernel(n, e, n_chunks1, 1, out_d, out_d + 16, 128,
                          r_num * n)
  acc1 = ek1(table1, k2d1, d2d1)
  return _comb(acc1, h, nt8, zero_resw, wa1s, 1, out_d, rows=400, resid=False)


# speedup vs baseline: 51.9136x; 1.0385x over previous
import functools
import math

import jax
import jax.numpy as jnp
from jax import lax
from jax.experimental import pallas as pl
from jax.experimental.pallas import tpu as pltpu
from jax.experimental.pallas import tpu_sc as plsc

NC = 2
NS = 16
NW = NC * NS
CH = 128

_HIGH = jax.lax.Precision.DEFAULT


def _dot(a, b):
  return jnp.dot(a, b, preferred_element_type=jnp.float32, precision=_HIGH)


def _acc_rows(n_nodes):
  rpt = (-(-n_nodes // NS) + 7) // 8 * 8
  return rpt, rpt * NS


def _proj_body(x_ref, nt8_ref, w_ref, o_ref):
  nj, T = w_ref.shape[0], w_ref.shape[1]
  x = x_ref[...]
  xm = [x * nt8_ref[:, t:t + 1] for t in range(T)]
  for j in range(nj):
    o = _dot(xm[0], w_ref[j, 0])
    for t in range(1, T):
      o = o + _dot(xm[t], w_ref[j, t])
    o_ref[j] = o.astype(o_ref.dtype)


def _proj(x, nt8, w9, rows, out_dtype=jnp.float32):
  n, d_in = x.shape
  nj, t_, _, d_out = w9.shape
  grid_i = n // rows
  out = pl.pallas_call(
      _proj_body,
      grid=(grid_i,),
      in_specs=[
          pl.BlockSpec((rows, d_in), lambda i: (i, 0)),
          pl.BlockSpec((rows, 8), lambda i: (i, 0)),
          pl.BlockSpec((nj, t_, d_in, d_out), lambda i: (0, 0, 0, 0)),
      ],
      out_specs=pl.BlockSpec((nj, rows, d_out), lambda i: (0, i, 0)),
      out_shape=jax.ShapeDtypeStruct((nj, n, d_out), out_dtype),
  )(x, nt8, w9)
  return out.reshape(nj * n, d_out)


def _make_edge_kernel(n_nodes, n_edges, n_chunks, heads, d, rw, ch, rn,
                      bf16_table=False):
  hd = heads * d
  cpt = n_chunks // NW
  rpt, n_pad = _acc_rows(n_nodes)
  assert n_chunks % NW == 0 and cpt % 2 == 0
  nvec = d // 16
  voff = rn
  qoff = 2 * rn
  nv = -(-n_edges // ch)
  tdt = jnp.bfloat16 if bf16_table else jnp.float32

  def body(table_hbm, kidx_hbm, dst_hbm, out_hbm,
           kidx_v, dst_v, vidx_r, qidx_r, kbuf, qbuf, vbuf, mbuf,
           acc, gsem0, gsem1, ssem0, ssem1):
    c = lax.axis_index("c")
    s = lax.axis_index("s")
    wid = s * NC + c
    t_ch0 = wid * cpt
    gsem = (gsem0, gsem1)
    ssem = (ssem0, ssem1)

    for i in range(ch):
      for j in range(rw // 16):
        mbuf[0, i, pl.ds(j * 16, 16)] = jnp.zeros((16,), jnp.float32)

    def zc(i, _):
      pltpu.sync_copy(mbuf.at[0], acc.at[pl.ds(s * rpt + i * ch, ch)])
      return 0
    lax.fori_loop(0, rpt // ch, zc, 0)
    if rpt % ch:
      rem = rpt % ch
      pltpu.sync_copy(mbuf.at[0, pl.ds(0, rem)],
                      acc.at[pl.ds(s * rpt + (rpt // ch) * ch, rem)])
    plsc.subcore_barrier()

    pltpu.sync_copy(kidx_hbm.at[pl.ds(t_ch0, cpt)], kidx_v)
    pltpu.sync_copy(dst_hbm.at[pl.ds(t_ch0, cpt)], dst_v)

    lane = lax.iota(jnp.int32, 16)

    def chunk_ok(ci):
      return jnp.logical_and(t_ch0 + ci < nv, ci < cpt)

    def start(ci, par):
      for j in range(ch // 16):
        sl = pl.ds(j * 16, 16)
        vidx_r[par, sl] = kidx_v[ci, sl] + voff
        qidx_r[par, sl] = dst_v[ci, sl] + qoff
      pltpu.async_copy(table_hbm.at[kidx_v.at[ci]], kbuf.at[par], gsem[par])
      pltpu.async_copy(table_hbm.at[qidx_r.at[par]], qbuf.at[par], gsem[par])
      pltpu.async_copy(table_hbm.at[vidx_r.at[par]], vbuf.at[par], gsem[par])

    def finish(ci, par):
      dummy = table_hbm.at[pl.ds(0, ch)]
      pltpu.make_async_copy(dummy, kbuf.at[par], gsem[par]).wait()
      pltpu.make_async_copy(dummy, qbuf.at[par], gsem[par]).wait()
      pltpu.make_async_copy(dummy, vbuf.at[par], gsem[par]).wait()

      @pl.when(ci >= 2)
      def _():
        pltpu.make_async_copy(mbuf.at[par], acc.at[dst_v.at[ci - 2]],
                              ssem[par]).wait()

      def one_edge(e):
        traws = []
        vparts = []
        for h in range(heads):
          if bf16_table:
            ka, kb = plsc.unpack(kbuf[par, e, pl.ds(h * d, d)],
                                 format=plsc.PackFormat.INTERLEAVED)
            qa, qb = plsc.unpack(qbuf[par, e, pl.ds(h * d, d)],
                                 format=plsc.PackFormat.INTERLEAVED)
            va, vb = plsc.unpack(vbuf[par, e, pl.ds(h * d, d)],
                                 format=plsc.PackFormat.INTERLEAVED)
            vparts.append((va, vb))
            p = ka * qa + kb * qb
          else:
            p = (kbuf[par, e, pl.ds(h * d, 16)] *
                 qbuf[par, e, pl.ds(h * d, 16)])
            for j in range(1, nvec):
              off = h * d + j * 16
              p = p + (kbuf[par, e, pl.ds(off, 16)] *
                       qbuf[par, e, pl.ds(off, 16)])
          traws.append(jnp.sum(p))
        denrow = jnp.zeros((16,), jnp.float32)
        for h in range(heads):
          exv = jnp.exp(jnp.full((16,), traws[h], jnp.float32))
          if bf16_table:
            va, vb = vparts[h]
            mbuf[par, e, pl.ds(h * d, 16)] = exv * va
            mbuf[par, e, pl.ds(h * d + 16, 16)] = exv * vb
          else:
            for j in range(nvec):
              off = h * d + j * 16
              mbuf[par, e, pl.ds(off, 16)] = exv * vbuf[par, e, pl.ds(off, 16)]
          denrow = denrow + jnp.where(lane == h, exv, 0.0)
        mbuf[par, e, pl.ds(hd, 16)] = denrow

      def edge_body(e2, _):
        for u in range(4):
          one_edge(e2 * 4 + u)
        return 0

      lax.fori_loop(0, ch // 4, edge_body, 0)
      pltpu.async_copy(mbuf.at[par], acc.at[dst_v.at[ci]], ssem[par],
                       add=True)

    @pl.when(chunk_ok(0))
    def _():
      start(0, 0)

    def pair_body(i2, _):
      ci0 = i2 * 2
      ci1 = ci0 + 1

      @pl.when(chunk_ok(ci1))
      def _():
        start(ci1, 1)

      @pl.when(chunk_ok(ci0))
      def _():
        finish(ci0, 0)

      @pl.when(chunk_ok(ci0 + 2))
      def _():
        start(ci0 + 2, 0)

      @pl.when(chunk_ok(ci1))
      def _():
        finish(ci1, 1)
      return 0

    lax.fori_loop(0, cpt // 2, pair_body, 0)

    for par in range(2):
      @pl.when(chunk_ok(par))
      def _():
        pltpu.make_async_copy(mbuf.at[par], acc.at[dst_v.at[par]],
                              ssem[par]).wait()
    plsc.subcore_barrier()
    pltpu.sync_copy(acc.at[pl.ds(s * rpt, rpt)],
                    out_hbm.at[c, pl.ds(s * rpt, rpt)])

  mesh = plsc.VectorSubcoreMesh(core_axis_name="c", subcore_axis_name="s",
                                num_cores=NC, num_subcores=NS)
  return pl.kernel(
      body,
      out_type=jax.ShapeDtypeStruct((NC, n_pad, rw), jnp.float32),
      mesh=mesh,
      compiler_params=pltpu.CompilerParams(needs_layout_passes=False,
                                           use_tc_tiling_on_sc=False),
      scratch_types=[
          pltpu.VMEM((cpt, ch), jnp.int32),
          pltpu.VMEM((cpt, ch), jnp.int32),
          pltpu.VMEM((2, ch), jnp.int32),
          pltpu.VMEM((2, ch), jnp.int32),
          pltpu.VMEM((2, ch, hd), tdt),
          pltpu.VMEM((2, ch, hd), tdt),
          pltpu.VMEM((2, ch, hd), tdt),
          pltpu.VMEM((2, ch, rw), jnp.float32),
          pltpu.VMEM_SHARED((n_pad, rw), jnp.float32),
          pltpu.SemaphoreType.DMA,
          pltpu.SemaphoreType.DMA,
          pltpu.SemaphoreType.DMA,
          pltpu.SemaphoreType.DMA,
      ],
  )


def _mid_body(acc_ref, x_ref, nt8_ref, resw_ref, wa_ref, w1_ref, o_ref,
              heads, d):
  T = wa_ref.shape[0]
  nj = w1_ref.shape[0]
  hd = heads * d
  s = acc_ref[0] + acc_ref[1]
  num = s[:, :hd]
  den = s[:, hd:hd + heads]
  inv = 1.0 / (den + 1e-9)
  rowi = lax.broadcasted_iota(jnp.int32, (heads, hd), 0)
  coli = lax.broadcasted_iota(jnp.int32, (heads, hd), 1)
  sel = (coli // d == rowi).astype(jnp.float32)
  h_att = num * _dot(inv, sel)
  masks = [nt8_ref[:, t:t + 1] for t in range(T)]
  h = _dot(h_att * masks[0], wa_ref[0])
  for t in range(1, T):
    h = h + _dot(h_att * masks[t], wa_ref[t])
  h = h + x_ref[...] * resw_ref[...]
  hm = [h * masks[t] for t in range(T)]
  for j in range(nj):
    o = _dot(hm[0], w1_ref[j, 0])
    for t in range(1, T):
      o = o + _dot(hm[t], w1_ref[j, t])
    o_ref[j] = o


def _mid(acc, x, nt8, resw, wa, w1, heads, d, rows):
  n, d_in = x.shape
  rw = acc.shape[2]
  nj, t_, _, d1 = w1.shape
  grid = n // rows
  body = functools.partial(_mid_body, heads=heads, d=d)
  out = pl.pallas_call(
      body,
      grid=(grid,),
      in_specs=[
          pl.BlockSpec((NC, rows, rw), lambda i: (0, i, 0)),
          pl.BlockSpec((rows, d_in), lambda i: (i, 0)),
          pl.BlockSpec((rows, 8), lambda i: (i, 0)),
          pl.BlockSpec((rows, 1), lambda i: (i, 0)),
          pl.BlockSpec((t_, d_in, d_in), lambda i: (0, 0, 0)),
          pl.BlockSpec((nj, t_, d_in, d1), lambda i: (0, 0, 0, 0)),
      ],
      out_specs=pl.BlockSpec((nj, rows, d1), lambda i: (0, i, 0)),
      out_shape=jax.ShapeDtypeStruct((nj, n, d1), jnp.float32),
  )(acc, x, nt8, resw, wa, w1)
  return out.reshape(nj * n, d1)


def _comb_body(acc_ref, x_ref, nt8_ref, resw_ref, wa_ref, o_ref, heads, d,
               resid):
  T = wa_ref.shape[0]
  hd = heads * d
  s = acc_ref[0] + acc_ref[1]
  num = s[:, :hd]
  den = s[:, hd:hd + heads]
  inv = 1.0 / (den + 1e-9)
  if heads > 1:
    rowi = lax.broadcasted_iota(jnp.int32, (heads, hd), 0)
    coli = lax.broadcasted_iota(jnp.int32, (heads, hd), 1)
    sel = (coli // d == rowi).astype(jnp.float32)
    invwide = _dot(inv, sel)
  else:
    invwide = inv
  h_att = num * invwide
  o = jnp.zeros(o_ref.shape, jnp.float32)
  for t in range(T):
    hm = h_att * nt8_ref[:, t:t + 1]
    o = o + _dot(hm, wa_ref[t])
  if resid:
    o = o + x_ref[...] * resw_ref[...]
  o_ref[...] = o


def _comb(acc, x, nt8, resw, wa, heads, d, rows, resid):
  n = x.shape[0]
  rw = acc.shape[2]
  t_, _, d_out = wa.shape
  grid = n // rows
  body = functools.partial(_comb_body, heads=heads, d=d, resid=resid)
  return pl.pallas_call(
      body,
      grid=(grid,),
      in_specs=[
          pl.BlockSpec((NC, rows, rw), lambda i: (0, i, 0)),
          pl.BlockSpec((rows, x.shape[1]), lambda i: (i, 0)),
          pl.BlockSpec((rows, 8), lambda i: (i, 0)),
          pl.BlockSpec((rows, 1), lambda i: (i, 0)),
          pl.BlockSpec((t_, wa.shape[1], d_out), lambda i: (0, 0, 0)),
      ],
      out_specs=pl.BlockSpec((rows, d_out), lambda i: (i, 0)),
      out_shape=jax.ShapeDtypeStruct((n, d_out), jnp.float32),
  )(acc, x, nt8, resw, wa)


def kernel(x, edge_index, node_type, edge_type, Wk0, Wq0, Wv0, ra0, rm0, rp0,
           Wa0, sk0, Wk1, Wq1, Wv1, ra1, rm1, rp1, Wa1, sk1):
  n, d_in = x.shape
  e = edge_index.shape[1]
  t_num = Wk0.shape[0]
  r_num = ra0.shape[0]
  h0, hid = ra0.shape[1], ra0.shape[2]
  out_d = ra1.shape[2]
  _, n_pad = _acc_rows(n)

  src = edge_index[0]
  dst = edge_index[1]

  def fused_tables(wk, wv, wq, ra, rm, rp, heads, dh):
    din = wk.shape[1]
    wkr = wk.reshape(t_num, din, heads, dh)
    wvr = wv.reshape(t_num, din, heads, dh)
    scale = 1.0 / math.sqrt(dh)
    wkra = jnp.einsum("tihd,rhdf,rh->rtihf", wkr, ra, rp).reshape(
        r_num, t_num, din, heads * dh)
    wvrm = jnp.einsum("tihd,rhdf->rtihf", wvr, rm).reshape(
        r_num, t_num, din, heads * dh)
    return jnp.concatenate(
        [wkra, wvrm, (wq * scale)[None]], axis=0)

  w9_0 = fused_tables(Wk0, Wv0, Wq0, ra0, rm0, rp0, h0, hid)
  w9_1 = fused_tables(Wk1, Wv1, Wq1, ra1, rm1, rp1, 1, out_d)
  perm = jnp.arange(h0 * hid).reshape(h0, hid // 2, 2).transpose(0, 2, 1)
  perm = perm.reshape(-1)
  sig0 = jax.nn.sigmoid(sk0)
  wa0s = (Wa0 * sig0[:, None, None])[:, perm, :]
  wa1s = Wa1 * jax.nn.sigmoid(sk1)[:, None, None]
  resw = (1.0 - sig0)[node_type][:, None]
  zero_resw = jnp.zeros((n, 1), jnp.float32)

  nt8 = (node_type[:, None] == jnp.arange(8, dtype=jnp.int32)[None, :]
         ).astype(jnp.float32)

  kidx = edge_type * n + src

  def chunked(ch):
    blk = NW * 8 * ch
    e_pad = ((e + blk - 1) // blk) * blk
    k2d = jnp.pad(kidx, [(0, e_pad - e)]).reshape(-1, ch)
    d2d = jnp.pad(dst, [(0, e_pad - e)],
                  constant_values=n_pad - 1).reshape(-1, ch)
    return k2d, d2d, e_pad // ch

  k2d0, d2d0, n_chunks0 = chunked(32)
  k2d1, d2d1, n_chunks1 = chunked(128)

  table0 = _proj(x, nt8, w9_0, rows=400, out_dtype=jnp.bfloat16)
  ek0 = _make_edge_kernel(n, e, n_chunks0, h0, hid, h0 * hid + 16, 32,
                          r_num * n, bf16_table=True)
  acc0 = ek0(table0, k2d0, d2d0)

  table1 = _mid(acc0, x, nt8, resw, wa0s, w9_1, h0, hid, rows=400)

  ek1 = _make_edge_kernel(n, e, n_chunks1, 1, out_d, out_d + 16, 128,
                          r_num * n)
  acc1 = ek1(table1, k2d1, d2d1)
  return _comb(acc1, x, nt8, zero_resw, wa1s, 1, out_d, rows=400, resid=False)
